# baseline (speedup 1.0000x reference)
import functools

import jax
import jax.numpy as jnp
from jax.experimental import pallas as pl
from jax.experimental.pallas import tpu as pltpu

N_NODES = 10000
N_PAD = 10240
HID = 256
HEADS = 4
NUM_GRAPHS = 64


def _mm_body(x_ref, w_ref, o_ref):
    o_ref[...] = jnp.dot(x_ref[...], w_ref[...],
                         preferred_element_type=jnp.float32)


def _matmul(x, w):
    n, k = x.shape
    k2, m = w.shape
    bn = 1000
    grid = (n // bn,)
    return pl.pallas_call(
        _mm_body,
        grid=grid,
        in_specs=[
            pl.BlockSpec((bn, k), lambda i: (i, 0)),
            pl.BlockSpec((k, m), lambda i: (0, 0)),
        ],
        out_specs=pl.BlockSpec((bn, m), lambda i: (i, 0)),
        out_shape=jax.ShapeDtypeStruct((n, m), jnp.float32),
    )(x, w)


def _gat_layer(h_in, src, dst, W, a_src, a_dst, b):
    N = h_in.shape[0]
    h = _matmul(h_in, W).reshape(N, HEADS, HID)
    alpha_src = (h * a_src).sum(-1)
    alpha_dst = (h * a_dst).sum(-1)
    e = alpha_src[src] + alpha_dst[dst]
    e = jax.nn.leaky_relu(e, negative_slope=0.2)
    emax = jax.ops.segment_max(e, dst, num_segments=N)
    e = jnp.exp(e - emax[dst])
    denom = jax.ops.segment_sum(e, dst, num_segments=N)
    alpha = e / (denom[dst] + 1e-16)
    msg = h[src] * alpha[:, :, None]
    out = jax.ops.segment_sum(msg, dst, num_segments=N)
    return jax.nn.relu(out.reshape(N, HEADS * HID) + b)


def kernel(x, edge_index, batch, W0, a_src0, a_dst0, b0, W1, a_src1, a_dst1,
           b1, W2, a_src2, a_dst2, b2, fc_W, fc_b):
    N = x.shape[0]
    loop = jnp.arange(N, dtype=edge_index.dtype)
    src = jnp.concatenate([edge_index[0], loop])
    dst = jnp.concatenate([edge_index[1], loop])
    h = x
    for (W, a_s, a_d, b) in ((W0, a_src0, a_dst0, b0),
                             (W1, a_src1, a_dst1, b1),
                             (W2, a_src2, a_dst2, b2)):
        h = _gat_layer(h, src, dst, W, a_s, a_d, b)
    sums = jax.ops.segment_sum(h, batch, num_segments=NUM_GRAPHS)
    counts = jax.ops.segment_sum(jnp.ones((N,), dtype=h.dtype), batch,
                                 num_segments=NUM_GRAPHS)
    pooled = sums / jnp.clip(counts, 1.0, None)[:, None]
    return pooled @ fc_W + fc_b


# speedup vs baseline: 8.2425x; 8.2425x over previous
import functools

import jax
import jax.numpy as jnp
from jax import lax
from jax.experimental import pallas as pl
from jax.experimental.pallas import tpu as pltpu
from jax.experimental.pallas import tpu_sc as plsc

N_NODES = 10000
NP = 10240
HID = 256
HEADS = 4
CH = HEADS * HID
NCHUNK = 8
CW = 128
NUM_GRAPHS = 64

TILES = 32
NPT = NP // TILES
EMAX = 6400
ET = 170000
BB = 2048
ET_PAD = 84 * BB
GB = 128

_mesh = plsc.VectorSubcoreMesh(core_axis_name="c", subcore_axis_name="s")


def _wid():
    return lax.axis_index("s") * 2 + lax.axis_index("c")


@functools.partial(
    pl.kernel,
    out_type=[
        jax.ShapeDtypeStruct((TILES, EMAX), jnp.int32),
        jax.ShapeDtypeStruct((TILES, EMAX), jnp.int32),
        jax.ShapeDtypeStruct((TILES, 16), jnp.int32),
    ],
    mesh=_mesh,
    compiler_params=pltpu.CompilerParams(needs_layout_passes=False),
    scratch_types=[
        pltpu.VMEM((BB,), jnp.int32),
        pltpu.VMEM((BB,), jnp.int32),
        pltpu.VMEM((EMAX,), jnp.int32),
        pltpu.VMEM((EMAX,), jnp.int32),
        pltpu.VMEM((16,), jnp.int32),
    ],
)
def _bucket(src_hbm, dst_hbm, esrc_hbm, edstl_hbm, ecnt_hbm,
            sbuf, dbuf, osrc, odstl, cntv):
    w = _wid()
    lo = w * NPT

    def zi(i, _):
        z = jnp.zeros((16,), jnp.int32)
        osrc[pl.ds(i * 16, 16)] = z
        odstl[pl.ds(i * 16, 16)] = z
        return 0

    lax.fori_loop(0, EMAX // 16, zi, 0)

    def blk(j, cnt):
        pltpu.sync_copy(src_hbm.at[pl.ds(j * BB, BB)], sbuf)
        pltpu.sync_copy(dst_hbm.at[pl.ds(j * BB, BB)], dbuf)

        def inner(i, cnt):
            sv = sbuf[pl.ds(i * 16, 16)]
            dv = dbuf[pl.ds(i * 16, 16)]
            m = (dv >= lo) & (dv < lo + NPT)
            cs = jnp.minimum(cnt, EMAX - 16)
            plsc.store_compressed(osrc.at[pl.ds(cs, 16)], sv, mask=m)
            plsc.store_compressed(odstl.at[pl.ds(cs, 16)], dv - lo, mask=m)
            pop = plsc.all_reduce_population_count(m)
            return cnt + pop[0]

        return lax.fori_loop(0, BB // 16, inner, cnt)

    cnt = lax.fori_loop(0, ET_PAD // BB, blk, 0)
    cntv[...] = jnp.zeros((16,), jnp.int32) + cnt
    pltpu.sync_copy(osrc, esrc_hbm.at[w])
    pltpu.sync_copy(odstl, edstl_hbm.at[w])
    pltpu.sync_copy(cntv, ecnt_hbm.at[w])


@functools.partial(
    pl.kernel,
    out_type=jax.ShapeDtypeStruct((NCHUNK, NP, CW), jnp.float32),
    mesh=_mesh,
    compiler_params=pltpu.CompilerParams(needs_layout_passes=False),
    scratch_types=[
        pltpu.VMEM((NP,), jnp.float32),
        pltpu.VMEM((NP,), jnp.float32),
        pltpu.VMEM((EMAX,), jnp.int32),
        pltpu.VMEM((EMAX,), jnp.int32),
        pltpu.VMEM((EMAX,), jnp.float32),
        pltpu.VMEM((NPT + 16,), jnp.float32),
        pltpu.VMEM((NPT + 16,), jnp.float32),
        pltpu.VMEM((NPT, CW), jnp.float32),
        pltpu.VMEM((2, GB, CW), jnp.float32),
        pltpu.VMEM((CW,), jnp.float32),
        pltpu.VMEM((16,), jnp.int32),
        pltpu.SemaphoreType.DMA((2,)),
    ],
)
def _aggregate(h8_hbm, asT_hbm, esrc_hbm, edstl_hbm, ecnt_hbm, b8_hbm,
               out8_hbm, asb, adb, srcb, dstlb, wbuf, den, inv, acc, grow,
               bbuf, cntv, gsem):
    w = _wid()
    lo = w * NPT
    pltpu.sync_copy(esrc_hbm.at[w], srcb)
    pltpu.sync_copy(edstl_hbm.at[w], dstlb)
    pltpu.sync_copy(ecnt_hbm.at[w], cntv)
    cnt = cntv[pl.ds(0, 16)][0]

    def za(i, _):
        r, v = i // 8, i % 8
        acc[r, pl.ds(v * 16, 16)] = jnp.zeros((16,), jnp.float32)
        return 0

    lax.fori_loop(0, NPT * 8, za, 0)

    def zd(i, _):
        den[pl.ds(i * 16, 16)] = jnp.zeros((16,), jnp.float32)
        return 0

    lax.fori_loop(0, NPT // 16 + 1, zd, 0)

    lane = lax.iota(jnp.int32, 16)
    one0 = (lane == 0).astype(jnp.float32)

    def head_body(hd, _):
        pltpu.sync_copy(asT_hbm.at[hd], asb)
        pltpu.sync_copy(asT_hbm.at[hd + 4], adb)

        def mx(i, mv):
            return jnp.maximum(mv, asb[pl.ds(i * 16, 16)])

        mv = lax.fori_loop(0, NP // 16, mx,
                           jnp.full((16,), -3e38, jnp.float32))
        Ms = jnp.max(mv)

        def wcomp(i, _):
            sv = srcb[pl.ds(i * 16, 16)]
            dvl = dstlb[pl.ds(i * 16, 16)]
            a_s = plsc.load_gather(asb, [sv])
            a_d = plsc.load_gather(adb, [dvl + lo])
            z = a_s + a_d
            e = jnp.where(z >= 0, z, 0.2 * z)
            zc = a_d + Ms
            cc = jnp.where(zc >= 0, zc, 0.2 * zc)
            wv = jnp.exp(e - cc)
            msk = (lane + i * 16) < cnt
            wbuf[pl.ds(i * 16, 16)] = jnp.where(msk, wv, 0.0)
            return 0

        lax.fori_loop(0, EMAX // 16, wcomp, 0)

        for sub in range(2):
            cidx = 2 * hd + sub
            nb = (cnt + GB - 1) // GB

            def gstart(j, par):
                pltpu.async_copy(
                    h8_hbm.at[cidx].at[srcb.at[pl.ds(j * GB, GB)]],
                    grow.at[par], gsem.at[par])

            def gwait(j, par):
                pltpu.make_async_copy(
                    h8_hbm.at[cidx].at[srcb.at[pl.ds(j * GB, GB)]],
                    grow.at[par], gsem.at[par]).wait()

            @pl.when(nb > 0)
            def _prime():
                gstart(0, 0)

            def eblk2(jj, _):
                for par in range(2):
                    j = 2 * jj + par

                    @pl.when(j < nb)
                    def _proc():
                        @pl.when(j + 1 < nb)
                        def _start_next():
                            gstart(j + 1, 1 - par)

                        gwait(j, par)
                        growp = grow.at[par]

                        def grp(kk, _):
                            base = j * GB + kk * 16
                            w16 = wbuf[pl.ds(base, 16)]
                            d16 = dstlb[pl.ds(base, 16)]
                            for q in range(16):
                                w_s = w16[q]
                                dl = d16[q]
                                k = kk * 16 + q
                                for v in range(8):
                                    g = growp[k, pl.ds(v * 16, 16)]
                                    a = acc[dl, pl.ds(v * 16, 16)]
                                    acc[dl, pl.ds(v * 16, 16)] = a + g * w_s
                                if sub == 0:
                                    dv = den[pl.ds(dl, 16)]
                                    den[pl.ds(dl, 16)] = dv + w_s * one0
                            return 0

                        lax.fori_loop(0, GB // 16, grp, 0)
                return 0

            lax.fori_loop(0, (EMAX // GB + 1) // 2, eblk2, 0)

            if sub == 0:
                def iv(i, _):
                    dv = den[pl.ds(i * 16, 16)]
                    inv[pl.ds(i * 16, 16)] = 1.0 / (dv + 1e-16)
                    den[pl.ds(i * 16, 16)] = jnp.zeros((16,), jnp.float32)
                    return 0

                lax.fori_loop(0, NPT // 16 + 1, iv, 0)

            pltpu.sync_copy(b8_hbm.at[cidx], bbuf)

            def rw(r, _):
                ivr = inv[pl.ds(r, 16)][0]
                for v in range(8):
                    o = acc[r, pl.ds(v * 16, 16)] * ivr + bbuf[pl.ds(v * 16, 16)]
                    acc[r, pl.ds(v * 16, 16)] = jnp.maximum(o, 0.0)
                return 0

            lax.fori_loop(0, NPT, rw, 0)
            pltpu.sync_copy(acc, out8_hbm.at[cidx, pl.ds(lo, NPT)])

            def za2(i, _):
                r, v = i // 8, i % 8
                acc[r, pl.ds(v * 16, 16)] = jnp.zeros((16,), jnp.float32)
                return 0

            lax.fori_loop(0, NPT * 8, za2, 0)
        return 0

    lax.fori_loop(0, HEADS, head_body, 0)


def _asel(af, hdiv):
    ki = lax.broadcasted_iota(jnp.int32, (CH, HEADS), 0)
    hi = lax.broadcasted_iota(jnp.int32, (CH, HEADS), 1)
    return jnp.where((ki // hdiv) == hi, af[:, None], 0.0)


def _mm0_body(x_ref, w_ref, asf_ref, adf_ref, h8_ref, asT_ref):
    x2 = x_ref[...]
    w2 = w_ref[...]
    res = jnp.dot(x2, w2, preferred_element_type=jnp.float32)
    for ci in range(NCHUNK):
        h8_ref[ci] = res[:, ci * CW:(ci + 1) * CW]
    asel_s = _asel(asf_ref[0], HID)
    asel_d = _asel(adf_ref[0], HID)
    wa_s = jnp.dot(w2, asel_s, preferred_element_type=jnp.float32)
    wa_d = jnp.dot(w2, asel_d, preferred_element_type=jnp.float32)
    dn = (((0,), (1,)), ((), ()))
    as_t = lax.dot_general(wa_s, x2, dn, preferred_element_type=jnp.float32)
    ad_t = lax.dot_general(wa_d, x2, dn, preferred_element_type=jnp.float32)
    asT_ref[...] = jnp.concatenate([as_t, ad_t], axis=0)


def _encoder_mm0(x, W, asf, adf):
    bn = 1024
    return pl.pallas_call(
        _mm0_body,
        grid=(NP // bn,),
        in_specs=[
            pl.BlockSpec((bn, 256), lambda i: (i, 0)),
            pl.BlockSpec((256, CH), lambda i: (0, 0)),
            pl.BlockSpec((1, CH), lambda i: (0, 0)),
            pl.BlockSpec((1, CH), lambda i: (0, 0)),
        ],
        out_specs=[
            pl.BlockSpec((NCHUNK, bn, CW), lambda i: (0, i, 0)),
            pl.BlockSpec((2 * HEADS, bn), lambda i: (0, i)),
        ],
        out_shape=[
            jax.ShapeDtypeStruct((NCHUNK, NP, CW), jnp.float32),
            jax.ShapeDtypeStruct((2 * HEADS, NP), jnp.float32),
        ],
    )(x, W, asf, adf)


def _mm_body(x_ref, w_ref, asf_ref, adf_ref, h8_ref, asT_ref):
    kc = pl.program_id(1)
    x2 = x_ref[0]
    w2 = w_ref[0]
    res = jnp.dot(x2, w2, preferred_element_type=jnp.float32)
    asel_s = _asel(asf_ref[0], HID)
    asel_d = _asel(adf_ref[0], HID)
    wa_s = jnp.dot(w2, asel_s, preferred_element_type=jnp.float32)
    wa_d = jnp.dot(w2, asel_d, preferred_element_type=jnp.float32)
    dn = (((0,), (1,)), ((), ()))
    as_t = lax.dot_general(wa_s, x2, dn, preferred_element_type=jnp.float32)
    ad_t = lax.dot_general(wa_d, x2, dn, preferred_element_type=jnp.float32)
    upd = jnp.concatenate([as_t, ad_t], axis=0)

    @pl.when(kc == 0)
    def _init():
        for ci in range(NCHUNK):
            h8_ref[ci] = res[:, ci * CW:(ci + 1) * CW]
        asT_ref[...] = upd

    @pl.when(kc > 0)
    def _accum():
        for ci in range(NCHUNK):
            h8_ref[ci] = h8_ref[ci] + res[:, ci * CW:(ci + 1) * CW]
        asT_ref[...] = asT_ref[...] + upd


def _encoder_mm(h8_in, W8, asf, adf):
    bn = 1024
    return pl.pallas_call(
        _mm_body,
        grid=(NP // bn, NCHUNK),
        in_specs=[
            pl.BlockSpec((1, bn, CW), lambda i, k: (k, i, 0)),
            pl.BlockSpec((1, CW, CH), lambda i, k: (k, 0, 0)),
            pl.BlockSpec((1, CH), lambda i, k: (0, 0)),
            pl.BlockSpec((1, CH), lambda i, k: (0, 0)),
        ],
        out_specs=[
            pl.BlockSpec((NCHUNK, bn, CW), lambda i, k: (0, i, 0)),
            pl.BlockSpec((2 * HEADS, bn), lambda i, k: (0, i)),
        ],
        out_shape=[
            jax.ShapeDtypeStruct((NCHUNK, NP, CW), jnp.float32),
            jax.ShapeDtypeStruct((2 * HEADS, NP), jnp.float32),
        ],
    )(h8_in, W8, asf, adf)


def _pool_body(h_ref, batch_ref, pooled_ref, counts_ref):
    c = pl.program_id(0)
    i = pl.program_id(1)
    bt = batch_ref[0, 0]
    oh = (bt[None, :] ==
          lax.broadcasted_iota(jnp.int32, (NUM_GRAPHS, 1024), 0)
          ).astype(jnp.float32)
    part = jnp.dot(oh, h_ref[0], preferred_element_type=jnp.float32)

    @pl.when(i == 0)
    def _init():
        pooled_ref[0] = part

    @pl.when(i > 0)
    def _accum():
        pooled_ref[0] = pooled_ref[0] + part

    @pl.when(c == 0)
    def _cnt():
        cp = jnp.sum(oh, axis=1)

        @pl.when(i == 0)
        def _ci():
            counts_ref[0] = cp

        @pl.when(i > 0)
        def _ca():
            counts_ref[0] = counts_ref[0] + cp


def _pool(h8, batch_r):
    bn = 1024
    return pl.pallas_call(
        _pool_body,
        grid=(NCHUNK, NP // bn),
        in_specs=[
            pl.BlockSpec((1, bn, CW), lambda c, i: (c, i, 0)),
            pl.BlockSpec((1, 1, bn), lambda c, i: (i, 0, 0)),
        ],
        out_specs=[
            pl.BlockSpec((1, NUM_GRAPHS, CW), lambda c, i: (c, 0, 0)),
            pl.BlockSpec((1, NUM_GRAPHS), lambda c, i: (0, 0)),
        ],
        out_shape=[
            jax.ShapeDtypeStruct((NCHUNK, NUM_GRAPHS, CW), jnp.float32),
            jax.ShapeDtypeStruct((1, NUM_GRAPHS), jnp.float32),
        ],
    )(h8, batch_r)


def _fc_body(pooled_ref, counts_ref, fcw_ref, fcb_ref, out_ref):
    inv = 1.0 / jnp.clip(counts_ref[0], 1.0, None)
    acc = jnp.zeros((NUM_GRAPHS, 10), jnp.float32)
    for c in range(NCHUNK):
        acc = acc + jnp.dot(pooled_ref[c] * inv[:, None], fcw_ref[c],
                            preferred_element_type=jnp.float32)
    out_ref[...] = acc + fcb_ref[0][None, :]


def _fc(pooled8, counts, fcw8, fcb):
    return pl.pallas_call(
        _fc_body,
        in_specs=[
            pl.BlockSpec((NCHUNK, NUM_GRAPHS, CW), lambda: (0, 0, 0)),
            pl.BlockSpec((1, NUM_GRAPHS), lambda: (0, 0)),
            pl.BlockSpec((NCHUNK, CW, 10), lambda: (0, 0, 0)),
            pl.BlockSpec((1, 10), lambda: (0, 0)),
        ],
        out_specs=pl.BlockSpec((NUM_GRAPHS, 10), lambda: (0, 0)),
        out_shape=jax.ShapeDtypeStruct((NUM_GRAPHS, 10), jnp.float32),
    )(pooled8, counts, fcw8, fcb)


def kernel(x, edge_index, batch, W0, a_src0, a_dst0, b0, W1, a_src1, a_dst1,
           b1, W2, a_src2, a_dst2, b2, fc_W, fc_b):
    N = x.shape[0]
    loop = jnp.arange(N, dtype=jnp.int32)
    src = jnp.concatenate([edge_index[0].astype(jnp.int32), loop])
    dst = jnp.concatenate([edge_index[1].astype(jnp.int32), loop])
    src_p = jnp.pad(src, (0, ET_PAD - ET))
    dst_p = jnp.pad(dst, (0, ET_PAD - ET), constant_values=1 << 30)

    esrc, edstl, ecnt = _bucket(src_p, dst_p)

    x_p = jnp.pad(x, ((0, NP - N), (0, 0)))
    h8, asT = _encoder_mm0(x_p, W0, a_src0.reshape(1, CH),
                           a_dst0.reshape(1, CH))
    h8 = _aggregate(h8, asT, esrc, edstl, ecnt, b0.reshape(NCHUNK, CW))
    for (W, a_s, a_d, b) in ((W1, a_src1, a_dst1, b1),
                             (W2, a_src2, a_dst2, b2)):
        h8, asT = _encoder_mm(h8, W.reshape(NCHUNK, CW, CH),
                              a_s.reshape(1, CH), a_d.reshape(1, CH))
        h8 = _aggregate(h8, asT, esrc, edstl, ecnt, b.reshape(NCHUNK, CW))

    batch_r = jnp.pad(batch.astype(jnp.int32), (0, NP - N),
                      constant_values=NUM_GRAPHS).reshape(NP // 1024, 1, 1024)
    pooled8, counts = _pool(h8, batch_r)
    return _fc(pooled8, counts, fc_W.reshape(NCHUNK, CW, 10),
               fc_b.reshape(1, 10))


# speedup vs baseline: 16.1543x; 1.9599x over previous
import functools

import jax
import jax.numpy as jnp
from jax import lax
from jax.experimental import pallas as pl
from jax.experimental.pallas import tpu as pltpu
from jax.experimental.pallas import tpu_sc as plsc

N_NODES = 10000
NP = 10240
HID = 256
HEADS = 4
CH = HEADS * HID
NCHUNK = 8
CW = 128
NUM_GRAPHS = 64

TILES = 32
NPT = NP // TILES
EMAX = 6400
ET = 170000
BB = 2048
ET_PAD = 84 * BB
GB = 128

_mesh = plsc.VectorSubcoreMesh(core_axis_name="c", subcore_axis_name="s")


def _wid():
    return lax.axis_index("s") * 2 + lax.axis_index("c")


@functools.partial(
    pl.kernel,
    out_type=[
        jax.ShapeDtypeStruct((TILES, EMAX), jnp.int32),
        jax.ShapeDtypeStruct((TILES, EMAX), jnp.int32),
        jax.ShapeDtypeStruct((TILES, 16), jnp.int32),
    ],
    mesh=_mesh,
    compiler_params=pltpu.CompilerParams(needs_layout_passes=False),
    scratch_types=[
        pltpu.VMEM((BB,), jnp.int32),
        pltpu.VMEM((BB,), jnp.int32),
        pltpu.VMEM((EMAX,), jnp.int32),
        pltpu.VMEM((EMAX,), jnp.int32),
        pltpu.VMEM((16,), jnp.int32),
    ],
)
def _bucket(src_hbm, dst_hbm, esrc_hbm, edstl_hbm, ecnt_hbm,
            sbuf, dbuf, osrc, odstl, cntv):
    w = _wid()
    lo = w * NPT

    def zi(i, _):
        z = jnp.zeros((16,), jnp.int32)
        osrc[pl.ds(i * 16, 16)] = z
        odstl[pl.ds(i * 16, 16)] = z
        return 0

    lax.fori_loop(0, EMAX // 16, zi, 0)

    def blk(j, cnt):
        pltpu.sync_copy(src_hbm.at[pl.ds(j * BB, BB)], sbuf)
        pltpu.sync_copy(dst_hbm.at[pl.ds(j * BB, BB)], dbuf)

        def inner(i, cnt):
            sv = sbuf[pl.ds(i * 16, 16)]
            dv = dbuf[pl.ds(i * 16, 16)]
            m = (dv >= lo) & (dv < lo + NPT)
            cs = jnp.minimum(cnt, EMAX - 16)
            plsc.store_compressed(osrc.at[pl.ds(cs, 16)], sv, mask=m)
            plsc.store_compressed(odstl.at[pl.ds(cs, 16)], dv - lo, mask=m)
            pop = plsc.all_reduce_population_count(m)
            return cnt + pop[0]

        return lax.fori_loop(0, BB // 16, inner, cnt)

    cnt = lax.fori_loop(0, ET_PAD // BB, blk, 0)
    cntv[...] = jnp.zeros((16,), jnp.int32) + cnt
    pltpu.sync_copy(osrc, esrc_hbm.at[w])
    pltpu.sync_copy(odstl, edstl_hbm.at[w])
    pltpu.sync_copy(cntv, ecnt_hbm.at[w])


@functools.partial(
    pl.kernel,
    out_type=jax.ShapeDtypeStruct((NCHUNK, NP, CW), jnp.float32),
    mesh=_mesh,
    compiler_params=pltpu.CompilerParams(needs_layout_passes=False),
    scratch_types=[
        pltpu.VMEM((NP,), jnp.float32),
        pltpu.VMEM((NP,), jnp.float32),
        pltpu.VMEM((EMAX,), jnp.int32),
        pltpu.VMEM((EMAX,), jnp.int32),
        pltpu.VMEM((EMAX,), jnp.float32),
        pltpu.VMEM((NPT + 16,), jnp.float32),
        pltpu.VMEM((NPT + 16,), jnp.float32),
        pltpu.VMEM((NPT, CW), jnp.float32),
        pltpu.VMEM((2, GB, CW), jnp.float32),
        pltpu.VMEM((CW,), jnp.float32),
        pltpu.VMEM((16,), jnp.int32),
        pltpu.SemaphoreType.DMA((2,)),
    ],
)
def _aggregate(h8_hbm, asT_hbm, esrc_hbm, edstl_hbm, ecnt_hbm, b8_hbm,
               out8_hbm, asb, adb, srcb, dstlb, wbuf, den, inv, acc, grow,
               bbuf, cntv, gsem):
    w = _wid()
    lo = w * NPT
    pltpu.sync_copy(esrc_hbm.at[w], srcb)
    pltpu.sync_copy(edstl_hbm.at[w], dstlb)
    pltpu.sync_copy(ecnt_hbm.at[w], cntv)
    cnt = cntv[pl.ds(0, 16)][0]

    def za(i, _):
        r, v = i // 8, i % 8
        acc[r, pl.ds(v * 16, 16)] = jnp.zeros((16,), jnp.float32)
        return 0

    lax.fori_loop(0, NPT * 8, za, 0)

    def zd(i, _):
        den[pl.ds(i * 16, 16)] = jnp.zeros((16,), jnp.float32)
        return 0

    lax.fori_loop(0, NPT // 16 + 1, zd, 0)

    lane = lax.iota(jnp.int32, 16)
    one0 = (lane == 0).astype(jnp.float32)

    def head_body(hd, _):
        pltpu.sync_copy(asT_hbm.at[hd], asb)
        pltpu.sync_copy(asT_hbm.at[hd + 4], adb)

        def mx(i, mv):
            return jnp.maximum(mv, asb[pl.ds(i * 16, 16)])

        mv = lax.fori_loop(0, NP // 16, mx,
                           jnp.full((16,), -3e38, jnp.float32))
        Ms = jnp.max(mv)

        def wcomp(i, _):
            sv = srcb[pl.ds(i * 16, 16)]
            dvl = dstlb[pl.ds(i * 16, 16)]
            a_s = plsc.load_gather(asb, [sv])
            a_d = plsc.load_gather(adb, [dvl + lo])
            z = a_s + a_d
            e = jnp.where(z >= 0, z, 0.2 * z)
            zc = a_d + Ms
            cc = jnp.where(zc >= 0, zc, 0.2 * zc)
            wv = jnp.exp(e - cc)
            msk = (lane + i * 16) < cnt
            wbuf[pl.ds(i * 16, 16)] = jnp.where(msk, wv, 0.0)
            return 0

        lax.fori_loop(0, EMAX // 16, wcomp, 0)

        for sub in range(2):
            cidx = 2 * hd + sub
            nb = (cnt + GB - 1) // GB

            def gstart(j, par):
                pltpu.async_copy(
                    h8_hbm.at[cidx].at[srcb.at[pl.ds(j * GB, GB)]],
                    grow.at[par], gsem.at[par])

            def gwait(j, par):
                pltpu.make_async_copy(
                    h8_hbm.at[cidx].at[srcb.at[pl.ds(j * GB, GB)]],
                    grow.at[par], gsem.at[par]).wait()

            @pl.when(nb > 0)
            def _prime():
                gstart(0, 0)

            def eblk2(jj, _):
                for par in range(2):
                    j = 2 * jj + par

                    @pl.when(j < nb)
                    def _proc():
                        @pl.when(j + 1 < nb)
                        def _start_next():
                            gstart(j + 1, 1 - par)

                        gwait(j, par)
                        growp = grow.at[par]

                        def grp(kk, _):
                            base = j * GB + kk * 16
                            w16 = wbuf[pl.ds(base, 16)]
                            d16 = dstlb[pl.ds(base, 16)]
                            for q in range(16):
                                w_s = w16[q]
                                dl = d16[q]
                                k = kk * 16 + q
                                gs = [growp[k, pl.ds(v * 16, 16)]
                                      for v in range(8)]
                                accs = [acc[dl, pl.ds(v * 16, 16)]
                                        for v in range(8)]
                                outs = [accs[v] + gs[v] * w_s
                                        for v in range(8)]
                                for v in range(8):
                                    acc[dl, pl.ds(v * 16, 16)] = outs[v]
                                if sub == 0:
                                    dv = den[pl.ds(dl, 16)]
                                    den[pl.ds(dl, 16)] = dv + w_s * one0
                            return 0

                        lax.fori_loop(0, GB // 16, grp, 0)
                return 0

            lax.fori_loop(0, (EMAX // GB + 1) // 2, eblk2, 0)

            if sub == 0:
                def iv(i, _):
                    dv = den[pl.ds(i * 16, 16)]
                    inv[pl.ds(i * 16, 16)] = 1.0 / (dv + 1e-16)
                    den[pl.ds(i * 16, 16)] = jnp.zeros((16,), jnp.float32)
                    return 0

                lax.fori_loop(0, NPT // 16 + 1, iv, 0)

            pltpu.sync_copy(b8_hbm.at[cidx], bbuf)

            def rw(r, _):
                ivr = inv[pl.ds(r, 16)][0]
                for v in range(8):
                    o = acc[r, pl.ds(v * 16, 16)] * ivr + bbuf[pl.ds(v * 16, 16)]
                    acc[r, pl.ds(v * 16, 16)] = jnp.maximum(o, 0.0)
                return 0

            lax.fori_loop(0, NPT, rw, 0)
            pltpu.sync_copy(acc, out8_hbm.at[cidx, pl.ds(lo, NPT)])

            def za2(i, _):
                r, v = i // 8, i % 8
                acc[r, pl.ds(v * 16, 16)] = jnp.zeros((16,), jnp.float32)
                return 0

            lax.fori_loop(0, NPT * 8, za2, 0)
        return 0

    lax.fori_loop(0, HEADS, head_body, 0)


def _asel(af, hdiv):
    ki = lax.broadcasted_iota(jnp.int32, (CH, HEADS), 0)
    hi = lax.broadcasted_iota(jnp.int32, (CH, HEADS), 1)
    return jnp.where((ki // hdiv) == hi, af[:, None], 0.0)


def _mm0_body(x_ref, w_ref, asf_ref, adf_ref, h8_ref, asT_ref):
    x2 = x_ref[...]
    w2 = w_ref[...]
    res = jnp.dot(x2, w2, preferred_element_type=jnp.float32)
    for ci in range(NCHUNK):
        h8_ref[ci] = res[:, ci * CW:(ci + 1) * CW]
    asel_s = _asel(asf_ref[0], HID)
    asel_d = _asel(adf_ref[0], HID)
    wa_s = jnp.dot(w2, asel_s, preferred_element_type=jnp.float32)
    wa_d = jnp.dot(w2, asel_d, preferred_element_type=jnp.float32)
    dn = (((0,), (1,)), ((), ()))
    as_t = lax.dot_general(wa_s, x2, dn, preferred_element_type=jnp.float32)
    ad_t = lax.dot_general(wa_d, x2, dn, preferred_element_type=jnp.float32)
    asT_ref[...] = jnp.concatenate([as_t, ad_t], axis=0)


def _encoder_mm0(x, W, asf, adf):
    bn = 1024
    return pl.pallas_call(
        _mm0_body,
        grid=(NP // bn,),
        in_specs=[
            pl.BlockSpec((bn, 256), lambda i: (i, 0)),
            pl.BlockSpec((256, CH), lambda i: (0, 0)),
            pl.BlockSpec((1, CH), lambda i: (0, 0)),
            pl.BlockSpec((1, CH), lambda i: (0, 0)),
        ],
        out_specs=[
            pl.BlockSpec((NCHUNK, bn, CW), lambda i: (0, i, 0)),
            pl.BlockSpec((2 * HEADS, bn), lambda i: (0, i)),
        ],
        out_shape=[
            jax.ShapeDtypeStruct((NCHUNK, NP, CW), jnp.float32),
            jax.ShapeDtypeStruct((2 * HEADS, NP), jnp.float32),
        ],
    )(x, W, asf, adf)


def _mm_body(x_ref, w_ref, asf_ref, adf_ref, h8_ref, asT_ref):
    kc = pl.program_id(1)
    x2 = x_ref[0]
    w2 = w_ref[0]
    res = jnp.dot(x2, w2, preferred_element_type=jnp.float32)
    asel_s = _asel(asf_ref[0], HID)
    asel_d = _asel(adf_ref[0], HID)
    wa_s = jnp.dot(w2, asel_s, preferred_element_type=jnp.float32)
    wa_d = jnp.dot(w2, asel_d, preferred_element_type=jnp.float32)
    dn = (((0,), (1,)), ((), ()))
    as_t = lax.dot_general(wa_s, x2, dn, preferred_element_type=jnp.float32)
    ad_t = lax.dot_general(wa_d, x2, dn, preferred_element_type=jnp.float32)
    upd = jnp.concatenate([as_t, ad_t], axis=0)

    @pl.when(kc == 0)
    def _init():
        for ci in range(NCHUNK):
            h8_ref[ci] = res[:, ci * CW:(ci + 1) * CW]
        asT_ref[...] = upd

    @pl.when(kc > 0)
    def _accum():
        for ci in range(NCHUNK):
            h8_ref[ci] = h8_ref[ci] + res[:, ci * CW:(ci + 1) * CW]
        asT_ref[...] = asT_ref[...] + upd


def _encoder_mm(h8_in, W8, asf, adf):
    bn = 1024
    return pl.pallas_call(
        _mm_body,
        grid=(NP // bn, NCHUNK),
        in_specs=[
            pl.BlockSpec((1, bn, CW), lambda i, k: (k, i, 0)),
            pl.BlockSpec((1, CW, CH), lambda i, k: (k, 0, 0)),
            pl.BlockSpec((1, CH), lambda i, k: (0, 0)),
            pl.BlockSpec((1, CH), lambda i, k: (0, 0)),
        ],
        out_specs=[
            pl.BlockSpec((NCHUNK, bn, CW), lambda i, k: (0, i, 0)),
            pl.BlockSpec((2 * HEADS, bn), lambda i, k: (0, i)),
        ],
        out_shape=[
            jax.ShapeDtypeStruct((NCHUNK, NP, CW), jnp.float32),
            jax.ShapeDtypeStruct((2 * HEADS, NP), jnp.float32),
        ],
    )(h8_in, W8, asf, adf)


def _pool_body(h_ref, batch_ref, pooled_ref, counts_ref):
    c = pl.program_id(0)
    i = pl.program_id(1)
    bt = batch_ref[0, 0]
    oh = (bt[None, :] ==
          lax.broadcasted_iota(jnp.int32, (NUM_GRAPHS, 1024), 0)
          ).astype(jnp.float32)
    part = jnp.dot(oh, h_ref[0], preferred_element_type=jnp.float32)

    @pl.when(i == 0)
    def _init():
        pooled_ref[0] = part

    @pl.when(i > 0)
    def _accum():
        pooled_ref[0] = pooled_ref[0] + part

    @pl.when(c == 0)
    def _cnt():
        cp = jnp.sum(oh, axis=1)

        @pl.when(i == 0)
        def _ci():
            counts_ref[0] = cp

        @pl.when(i > 0)
        def _ca():
            counts_ref[0] = counts_ref[0] + cp


def _pool(h8, batch_r):
    bn = 1024
    return pl.pallas_call(
        _pool_body,
        grid=(NCHUNK, NP // bn),
        in_specs=[
            pl.BlockSpec((1, bn, CW), lambda c, i: (c, i, 0)),
            pl.BlockSpec((1, 1, bn), lambda c, i: (i, 0, 0)),
        ],
        out_specs=[
            pl.BlockSpec((1, NUM_GRAPHS, CW), lambda c, i: (c, 0, 0)),
            pl.BlockSpec((1, NUM_GRAPHS), lambda c, i: (0, 0)),
        ],
        out_shape=[
            jax.ShapeDtypeStruct((NCHUNK, NUM_GRAPHS, CW), jnp.float32),
            jax.ShapeDtypeStruct((1, NUM_GRAPHS), jnp.float32),
        ],
    )(h8, batch_r)


def _fc_body(pooled_ref, counts_ref, fcw_ref, fcb_ref, out_ref):
    inv = 1.0 / jnp.clip(counts_ref[0], 1.0, None)
    acc = jnp.zeros((NUM_GRAPHS, 10), jnp.float32)
    for c in range(NCHUNK):
        acc = acc + jnp.dot(pooled_ref[c] * inv[:, None], fcw_ref[c],
                            preferred_element_type=jnp.float32)
    out_ref[...] = acc + fcb_ref[0][None, :]


def _fc(pooled8, counts, fcw8, fcb):
    return pl.pallas_call(
        _fc_body,
        in_specs=[
            pl.BlockSpec((NCHUNK, NUM_GRAPHS, CW), lambda: (0, 0, 0)),
            pl.BlockSpec((1, NUM_GRAPHS), lambda: (0, 0)),
            pl.BlockSpec((NCHUNK, CW, 10), lambda: (0, 0, 0)),
            pl.BlockSpec((1, 10), lambda: (0, 0)),
        ],
        out_specs=pl.BlockSpec((NUM_GRAPHS, 10), lambda: (0, 0)),
        out_shape=jax.ShapeDtypeStruct((NUM_GRAPHS, 10), jnp.float32),
    )(pooled8, counts, fcw8, fcb)


def kernel(x, edge_index, batch, W0, a_src0, a_dst0, b0, W1, a_src1, a_dst1,
           b1, W2, a_src2, a_dst2, b2, fc_W, fc_b):
    N = x.shape[0]
    loop = jnp.arange(N, dtype=jnp.int32)
    src = jnp.concatenate([edge_index[0].astype(jnp.int32), loop])
    dst = jnp.concatenate([edge_index[1].astype(jnp.int32), loop])
    src_p = jnp.pad(src, (0, ET_PAD - ET))
    dst_p = jnp.pad(dst, (0, ET_PAD - ET), constant_values=1 << 30)

    esrc, edstl, ecnt = _bucket(src_p, dst_p)

    x_p = jnp.pad(x, ((0, NP - N), (0, 0)))
    h8, asT = _encoder_mm0(x_p, W0, a_src0.reshape(1, CH),
                           a_dst0.reshape(1, CH))
    h8 = _aggregate(h8, asT, esrc, edstl, ecnt, b0.reshape(NCHUNK, CW))
    for (W, a_s, a_d, b) in ((W1, a_src1, a_dst1, b1),
                             (W2, a_src2, a_dst2, b2)):
        h8, asT = _encoder_mm(h8, W.reshape(NCHUNK, CW, CH),
                              a_s.reshape(1, CH), a_d.reshape(1, CH))
        h8 = _aggregate(h8, asT, esrc, edstl, ecnt, b.reshape(NCHUNK, CW))

    batch_r = jnp.pad(batch.astype(jnp.int32), (0, NP - N),
                      constant_values=NUM_GRAPHS).reshape(NP // 1024, 1, 1024)
    pooled8, counts = _pool(h8, batch_r)
    return _fc(pooled8, counts, fc_W.reshape(NCHUNK, CW, 10),
               fc_b.reshape(1, 10))


# speedup vs baseline: 17.0246x; 1.0539x over previous
import functools

import jax
import jax.numpy as jnp
import numpy as np
from jax import lax
from jax.experimental import pallas as pl
from jax.experimental.pallas import tpu as pltpu
from jax.experimental.pallas import tpu_sc as plsc

N_NODES = 10000
NP = 10240
HID = 256
HEADS = 4
CH = HEADS * HID
NCHUNK = 8
CW = 128
NUM_GRAPHS = 64

TILES = 32
NPT = NP // TILES
EMAX = 6400
ET = 170000
BB = 2048
ET_PAD = 84 * BB
GB = 128

_PERM = np.arange(CH).reshape(-1, 2, 16).transpose(0, 2, 1).reshape(-1)

_mesh = plsc.VectorSubcoreMesh(core_axis_name="c", subcore_axis_name="s")


def _wid():
    return lax.axis_index("s") * 2 + lax.axis_index("c")


@functools.partial(
    pl.kernel,
    out_type=[
        jax.ShapeDtypeStruct((TILES, EMAX), jnp.int32),
        jax.ShapeDtypeStruct((TILES, EMAX), jnp.int32),
        jax.ShapeDtypeStruct((TILES, 16), jnp.int32),
    ],
    mesh=_mesh,
    compiler_params=pltpu.CompilerParams(needs_layout_passes=False),
    scratch_types=[
        pltpu.VMEM((BB,), jnp.int32),
        pltpu.VMEM((BB,), jnp.int32),
        pltpu.VMEM((EMAX,), jnp.int32),
        pltpu.VMEM((EMAX,), jnp.int32),
        pltpu.VMEM((16,), jnp.int32),
    ],
)
def _bucket(src_hbm, dst_hbm, esrc_hbm, edstl_hbm, ecnt_hbm,
            sbuf, dbuf, osrc, odstl, cntv):
    w = _wid()
    lo = w * NPT

    def zi(i, _):
        z = jnp.zeros((16,), jnp.int32)
        osrc[pl.ds(i * 16, 16)] = z
        odstl[pl.ds(i * 16, 16)] = z
        return 0

    lax.fori_loop(0, EMAX // 16, zi, 0)

    def blk(j, cnt):
        pltpu.sync_copy(src_hbm.at[pl.ds(j * BB, BB)], sbuf)
        pltpu.sync_copy(dst_hbm.at[pl.ds(j * BB, BB)], dbuf)

        def inner(i, cnt):
            sv = sbuf[pl.ds(i * 16, 16)]
            dv = dbuf[pl.ds(i * 16, 16)]
            m = (dv >= lo) & (dv < lo + NPT)
            cs = jnp.minimum(cnt, EMAX - 16)
            plsc.store_compressed(osrc.at[pl.ds(cs, 16)], sv, mask=m)
            plsc.store_compressed(odstl.at[pl.ds(cs, 16)], dv - lo, mask=m)
            pop = plsc.all_reduce_population_count(m)
            return cnt + pop[0]

        return lax.fori_loop(0, BB // 16, inner, cnt)

    cnt = lax.fori_loop(0, ET_PAD // BB, blk, 0)
    cntv[...] = jnp.zeros((16,), jnp.int32) + cnt
    pltpu.sync_copy(osrc, esrc_hbm.at[w])
    pltpu.sync_copy(odstl, edstl_hbm.at[w])
    pltpu.sync_copy(cntv, ecnt_hbm.at[w])


@functools.partial(
    pl.kernel,
    out_type=jax.ShapeDtypeStruct((NCHUNK, NP, CW), jnp.float32),
    mesh=_mesh,
    compiler_params=pltpu.CompilerParams(needs_layout_passes=False),
    scratch_types=[
        pltpu.VMEM((NP,), jnp.float32),
        pltpu.VMEM((NP,), jnp.float32),
        pltpu.VMEM((EMAX,), jnp.int32),
        pltpu.VMEM((EMAX,), jnp.int32),
        pltpu.VMEM((EMAX,), jnp.float32),
        pltpu.VMEM((NPT + 16,), jnp.float32),
        pltpu.VMEM((NPT + 16,), jnp.float32),
        pltpu.VMEM((NPT, CW), jnp.float32),
        pltpu.VMEM((2, GB, CW), jnp.float32),
        pltpu.VMEM((CW,), jnp.float32),
        pltpu.VMEM((16,), jnp.int32),
        pltpu.SemaphoreType.DMA((2,)),
    ],
)
def _aggregate(h8_hbm, asT_hbm, esrc_hbm, edstl_hbm, ecnt_hbm, b8_hbm,
               out8_hbm, asb, adb, srcb, dstlb, wbuf, den, inv, acc, grow,
               bbuf, cntv, gsem):
    w = _wid()
    lo = w * NPT
    pltpu.sync_copy(esrc_hbm.at[w], srcb)
    pltpu.sync_copy(edstl_hbm.at[w], dstlb)
    pltpu.sync_copy(ecnt_hbm.at[w], cntv)
    cnt = cntv[pl.ds(0, 16)][0]

    def za(i, _):
        r, v = i // 8, i % 8
        acc[r, pl.ds(v * 16, 16)] = jnp.zeros((16,), jnp.float32)
        return 0

    lax.fori_loop(0, NPT * 8, za, 0)

    def zd(i, _):
        den[pl.ds(i * 16, 16)] = jnp.zeros((16,), jnp.float32)
        return 0

    lax.fori_loop(0, NPT // 16 + 1, zd, 0)

    lane = lax.iota(jnp.int32, 16)
    one0 = (lane == 0).astype(jnp.float32)

    def head_body(hd, _):
        pltpu.sync_copy(asT_hbm.at[hd], asb)
        pltpu.sync_copy(asT_hbm.at[hd + 4], adb)

        def mx(i, mv):
            return jnp.maximum(mv, asb[pl.ds(i * 16, 16)])

        mv = lax.fori_loop(0, NP // 16, mx,
                           jnp.full((16,), -3e38, jnp.float32))
        Ms = jnp.max(mv)

        def wcomp(i, _):
            sv = srcb[pl.ds(i * 16, 16)]
            dvl = dstlb[pl.ds(i * 16, 16)]
            a_s = plsc.load_gather(asb, [sv])
            a_d = plsc.load_gather(adb, [dvl + lo])
            z = a_s + a_d
            e = jnp.where(z >= 0, z, 0.2 * z)
            zc = a_d + Ms
            cc = jnp.where(zc >= 0, zc, 0.2 * zc)
            wv = jnp.exp(e - cc)
            msk = (lane + i * 16) < cnt
            wv = jnp.where(msk, wv, 0.0)
            wbuf[pl.ds(i * 16, 16)] = wv
            plsc.addupdate_scatter(den, [dvl], wv)
            return 0

        lax.fori_loop(0, EMAX // 16, wcomp, 0)

        def iv(i, _):
            dv = den[pl.ds(i * 16, 16)]
            inv[pl.ds(i * 16, 16)] = 1.0 / (dv + 1e-16)
            den[pl.ds(i * 16, 16)] = jnp.zeros((16,), jnp.float32)
            return 0

        lax.fori_loop(0, NPT // 16 + 1, iv, 0)

        for sub in range(2):
            cidx = 2 * hd + sub
            nb = (cnt + GB - 1) // GB

            def gstart(j, par):
                pltpu.async_copy(
                    h8_hbm.at[cidx].at[srcb.at[pl.ds(j * GB, GB)]],
                    grow.at[par], gsem.at[par])

            def gwait(j, par):
                pltpu.make_async_copy(
                    h8_hbm.at[cidx].at[srcb.at[pl.ds(j * GB, GB)]],
                    grow.at[par], gsem.at[par]).wait()

            @pl.when(nb > 0)
            def _prime():
                gstart(0, 0)

            def eblk2(jj, _):
                for par in range(2):
                    j = 2 * jj + par

                    @pl.when(j < nb)
                    def _proc():
                        @pl.when(j + 1 < nb)
                        def _start_next():
                            gstart(j + 1, 1 - par)

                        gwait(j, par)
                        growp = grow.at[par]

                        def grp(kk, _):
                            base = j * GB + kk * 16
                            w16 = wbuf[pl.ds(base, 16)]
                            d16 = dstlb[pl.ds(base, 16)]
                            for q in range(16):
                                w_s = w16[q]
                                dl = d16[q]
                                k = kk * 16 + q
                                gs = [growp[k, pl.ds(v * 16, 16)]
                                      for v in range(8)]
                                accs = [acc[dl, pl.ds(v * 16, 16)]
                                        for v in range(8)]
                                outs = [accs[v] + gs[v] * w_s
                                        for v in range(8)]
                                for v in range(8):
                                    acc[dl, pl.ds(v * 16, 16)] = outs[v]
                            return 0

                        lax.fori_loop(0, GB // 16, grp, 0)
                return 0

            lax.fori_loop(0, (EMAX // GB + 1) // 2, eblk2, 0)

            pltpu.sync_copy(b8_hbm.at[cidx], bbuf)

            def rw(r, _):
                ivr = inv[pl.ds(r, 16)][0]
                for v in range(8):
                    o = acc[r, pl.ds(v * 16, 16)] * ivr + bbuf[pl.ds(v * 16, 16)]
                    acc[r, pl.ds(v * 16, 16)] = jnp.maximum(o, 0.0)
                return 0

            lax.fori_loop(0, NPT, rw, 0)
            pltpu.sync_copy(acc, out8_hbm.at[cidx, pl.ds(lo, NPT)])

            def za2(i, _):
                r, v = i // 8, i % 8
                acc[r, pl.ds(v * 16, 16)] = jnp.zeros((16,), jnp.float32)
                return 0

            lax.fori_loop(0, NPT * 8, za2, 0)
        return 0

    lax.fori_loop(0, HEADS, head_body, 0)


def _asel(af, hdiv):
    ki = lax.broadcasted_iota(jnp.int32, (CH, HEADS), 0)
    hi = lax.broadcasted_iota(jnp.int32, (CH, HEADS), 1)
    return jnp.where((ki // hdiv) == hi, af[:, None], 0.0)


def _mm0_body(x_ref, w_ref, asf_ref, adf_ref, h8_ref, asT_ref):
    x2 = x_ref[...].astype(jnp.bfloat16)
    w2 = w_ref[...].astype(jnp.bfloat16)
    res = jnp.dot(x2, w2, preferred_element_type=jnp.float32)
    for ci in range(NCHUNK):
        h8_ref[ci] = res[:, ci * CW:(ci + 1) * CW]
    asel_s = _asel(asf_ref[0], HID)
    asel_d = _asel(adf_ref[0], HID)
    wa_s = jnp.dot(w2, asel_s, preferred_element_type=jnp.float32)
    wa_d = jnp.dot(w2, asel_d, preferred_element_type=jnp.float32)
    dn = (((0,), (1,)), ((), ()))
    as_t = lax.dot_general(wa_s, x2, dn, preferred_element_type=jnp.float32)
    ad_t = lax.dot_general(wa_d, x2, dn, preferred_element_type=jnp.float32)
    asT_ref[...] = jnp.concatenate([as_t, ad_t], axis=0)


def _encoder_mm0(x, W, asf, adf):
    bn = 1024
    return pl.pallas_call(
        _mm0_body,
        grid=(NP // bn,),
        in_specs=[
            pl.BlockSpec((bn, 256), lambda i: (i, 0)),
            pl.BlockSpec((256, CH), lambda i: (0, 0)),
            pl.BlockSpec((1, CH), lambda i: (0, 0)),
            pl.BlockSpec((1, CH), lambda i: (0, 0)),
        ],
        out_specs=[
            pl.BlockSpec((NCHUNK, bn, CW), lambda i: (0, i, 0)),
            pl.BlockSpec((2 * HEADS, bn), lambda i: (0, i)),
        ],
        out_shape=[
            jax.ShapeDtypeStruct((NCHUNK, NP, CW), jnp.float32),
            jax.ShapeDtypeStruct((2 * HEADS, NP), jnp.float32),
        ],
    )(x, W, asf, adf)


def _mm_body(x_ref, w_ref, asf_ref, adf_ref, h8_ref, asT_ref, sc_ref):
    kc = pl.program_id(1)
    x2 = x_ref[0].astype(jnp.bfloat16)
    w2 = w_ref[0].astype(jnp.bfloat16)
    res = jnp.dot(x2, w2, preferred_element_type=jnp.float32)
    asel_s = _asel(asf_ref[0], HID)
    asel_d = _asel(adf_ref[0], HID)
    wa_s = jnp.dot(w2, asel_s, preferred_element_type=jnp.float32)
    wa_d = jnp.dot(w2, asel_d, preferred_element_type=jnp.float32)
    dn = (((0,), (1,)), ((), ()))
    as_t = lax.dot_general(wa_s, x2, dn, preferred_element_type=jnp.float32)
    ad_t = lax.dot_general(wa_d, x2, dn, preferred_element_type=jnp.float32)
    upd = jnp.concatenate([as_t, ad_t], axis=0)

    @pl.when(kc == 0)
    def _init():
        sc_ref[...] = res
        asT_ref[...] = upd

    @pl.when(kc > 0)
    def _accum():
        sc_ref[...] = sc_ref[...] + res
        asT_ref[...] = asT_ref[...] + upd

    @pl.when(kc == NCHUNK - 1)
    def _emit():
        tot = sc_ref[...]
        for ci in range(NCHUNK):
            h8_ref[ci] = tot[:, ci * CW:(ci + 1) * CW]


def _encoder_mm(h8_in, W8, asf, adf):
    bn = 1024
    return pl.pallas_call(
        _mm_body,
        grid=(NP // bn, NCHUNK),
        in_specs=[
            pl.BlockSpec((1, bn, CW), lambda i, k: (k, i, 0)),
            pl.BlockSpec((1, CW, CH), lambda i, k: (k, 0, 0)),
            pl.BlockSpec((1, CH), lambda i, k: (0, 0)),
            pl.BlockSpec((1, CH), lambda i, k: (0, 0)),
        ],
        out_specs=[
            pl.BlockSpec((NCHUNK, bn, CW), lambda i, k: (0, i, 0)),
            pl.BlockSpec((2 * HEADS, bn), lambda i, k: (0, i)),
        ],
        out_shape=[
            jax.ShapeDtypeStruct((NCHUNK, NP, CW), jnp.float32),
            jax.ShapeDtypeStruct((2 * HEADS, NP), jnp.float32),
        ],
        scratch_shapes=[pltpu.VMEM((bn, CH), jnp.float32)],
    )(h8_in, W8, asf, adf)


def _pool_body(h_ref, batch_ref, pooled_ref, counts_ref):
    c = pl.program_id(0)
    i = pl.program_id(1)
    bt = batch_ref[0, 0]
    oh = (bt[None, :] ==
          lax.broadcasted_iota(jnp.int32, (NUM_GRAPHS, 1024), 0)
          ).astype(jnp.float32)
    part = jnp.dot(oh, h_ref[0], preferred_element_type=jnp.float32)

    @pl.when(i == 0)
    def _init():
        pooled_ref[0] = part

    @pl.when(i > 0)
    def _accum():
        pooled_ref[0] = pooled_ref[0] + part

    @pl.when(c == 0)
    def _cnt():
        cp = jnp.sum(oh, axis=1)

        @pl.when(i == 0)
        def _ci():
            counts_ref[0] = cp

        @pl.when(i > 0)
        def _ca():
            counts_ref[0] = counts_ref[0] + cp


def _pool(h8, batch_r):
    bn = 1024
    return pl.pallas_call(
        _pool_body,
        grid=(NCHUNK, NP // bn),
        in_specs=[
            pl.BlockSpec((1, bn, CW), lambda c, i: (c, i, 0)),
            pl.BlockSpec((1, 1, bn), lambda c, i: (i, 0, 0)),
        ],
        out_specs=[
            pl.BlockSpec((1, NUM_GRAPHS, CW), lambda c, i: (c, 0, 0)),
            pl.BlockSpec((1, NUM_GRAPHS), lambda c, i: (0, 0)),
        ],
        out_shape=[
            jax.ShapeDtypeStruct((NCHUNK, NUM_GRAPHS, CW), jnp.float32),
            jax.ShapeDtypeStruct((1, NUM_GRAPHS), jnp.float32),
        ],
    )(h8, batch_r)


def _fc_body(pooled_ref, counts_ref, fcw_ref, fcb_ref, out_ref):
    inv = 1.0 / jnp.clip(counts_ref[0], 1.0, None)
    acc = jnp.zeros((NUM_GRAPHS, 10), jnp.float32)
    for c in range(NCHUNK):
        acc = acc + jnp.dot(pooled_ref[c] * inv[:, None], fcw_ref[c],
                            preferred_element_type=jnp.float32)
    out_ref[...] = acc + fcb_ref[0][None, :]


def _fc(pooled8, counts, fcw8, fcb):
    return pl.pallas_call(
        _fc_body,
        in_specs=[
            pl.BlockSpec((NCHUNK, NUM_GRAPHS, CW), lambda: (0, 0, 0)),
            pl.BlockSpec((1, NUM_GRAPHS), lambda: (0, 0)),
            pl.BlockSpec((NCHUNK, CW, 10), lambda: (0, 0, 0)),
            pl.BlockSpec((1, 10), lambda: (0, 0)),
        ],
        out_specs=pl.BlockSpec((NUM_GRAPHS, 10), lambda: (0, 0)),
        out_shape=jax.ShapeDtypeStruct((NUM_GRAPHS, 10), jnp.float32),
    )(pooled8, counts, fcw8, fcb)


def kernel(x, edge_index, batch, W0, a_src0, a_dst0, b0, W1, a_src1, a_dst1,
           b1, W2, a_src2, a_dst2, b2, fc_W, fc_b):
    N = x.shape[0]
    loop = jnp.arange(N, dtype=jnp.int32)
    src = jnp.concatenate([edge_index[0].astype(jnp.int32), loop])
    dst = jnp.concatenate([edge_index[1].astype(jnp.int32), loop])
    src_p = jnp.pad(src, (0, ET_PAD - ET))
    dst_p = jnp.pad(dst, (0, ET_PAD - ET), constant_values=1 << 30)

    esrc, edstl, ecnt = _bucket(src_p, dst_p)

    x_p = jnp.pad(x, ((0, NP - N), (0, 0)))
    h8, asT = _encoder_mm0(x_p, W0, a_src0.reshape(1, CH),
                           a_dst0.reshape(1, CH))
    h8 = _aggregate(h8, asT, esrc, edstl, ecnt, b0.reshape(NCHUNK, CW))
    for (W, a_s, a_d, b) in ((W1, a_src1, a_dst1, b1),
                             (W2, a_src2, a_dst2, b2)):
        h8, asT = _encoder_mm(h8, W.reshape(NCHUNK, CW, CH),
                              a_s.reshape(1, CH), a_d.reshape(1, CH))
        h8 = _aggregate(h8, asT, esrc, edstl, ecnt, b.reshape(NCHUNK, CW))

    batch_r = jnp.pad(batch.astype(jnp.int32), (0, NP - N),
                      constant_values=NUM_GRAPHS).reshape(NP // 1024, 1, 1024)
    pooled8, counts = _pool(h8, batch_r)
    return _fc(pooled8, counts, fc_W.reshape(NCHUNK, CW, 10),
               fc_b.reshape(1, 10))


# speedup vs baseline: 18.3715x; 1.0791x over previous
import functools

import jax
import jax.numpy as jnp
import numpy as np
from jax import lax
from jax.experimental import pallas as pl
from jax.experimental.pallas import tpu as pltpu
from jax.experimental.pallas import tpu_sc as plsc

N_NODES = 10000
NP = 10240
HID = 256
HEADS = 4
CH = HEADS * HID
NCHUNK = 8
CW = 128
NUM_GRAPHS = 64

TILES = 32
NPT = NP // TILES
EMAX = 6400
ET = 170000
BB = 2048
ET_PAD = 84 * BB
GB = 128

_PERM = np.arange(CH).reshape(-1, 2, 16).transpose(0, 2, 1).reshape(-1)

_mesh = plsc.VectorSubcoreMesh(core_axis_name="c", subcore_axis_name="s")


def _wid():
    return lax.axis_index("s") * 2 + lax.axis_index("c")


@functools.partial(
    pl.kernel,
    out_type=[
        jax.ShapeDtypeStruct((TILES, EMAX), jnp.int32),
        jax.ShapeDtypeStruct((TILES, EMAX), jnp.int32),
        jax.ShapeDtypeStruct((TILES, 16), jnp.int32),
        jax.ShapeDtypeStruct((TILES, NPT + 16), jnp.int32),
    ],
    mesh=_mesh,
    compiler_params=pltpu.CompilerParams(needs_layout_passes=False),
    scratch_types=[
        pltpu.VMEM((BB,), jnp.int32),
        pltpu.VMEM((BB,), jnp.int32),
        pltpu.VMEM((EMAX,), jnp.int32),
        pltpu.VMEM((EMAX,), jnp.int32),
        pltpu.VMEM((EMAX + 16,), jnp.int32),
        pltpu.VMEM((EMAX + 16,), jnp.int32),
        pltpu.VMEM((NPT + 16,), jnp.int32),
        pltpu.VMEM((NPT + 16,), jnp.int32),
        pltpu.VMEM((16,), jnp.int32),
    ],
)
def _bucket(src_hbm, dst_hbm, esrc_hbm, edstl_hbm, ecnt_hbm, eoff_hbm,
            sbuf, dbuf, osrc, odstl, ssrc, sdst, cnts, offb, cntv):
    w = _wid()
    lo = w * NPT
    lane = lax.iota(jnp.int32, 16)
    one0i = (lane == 0).astype(jnp.int32)

    def zi(i, _):
        osrc[pl.ds(i * 16, 16)] = jnp.zeros((16,), jnp.int32)
        odstl[pl.ds(i * 16, 16)] = jnp.zeros((16,), jnp.int32) + NPT
        return 0

    lax.fori_loop(0, EMAX // 16, zi, 0)

    def zc(i, _):
        cnts[pl.ds(i * 16, 16)] = jnp.zeros((16,), jnp.int32)
        return 0

    lax.fori_loop(0, (NPT + 16) // 16, zc, 0)

    def blk(j, cnt):
        pltpu.sync_copy(src_hbm.at[pl.ds(j * BB, BB)], sbuf)
        pltpu.sync_copy(dst_hbm.at[pl.ds(j * BB, BB)], dbuf)

        def inner(i, cnt):
            sv = sbuf[pl.ds(i * 16, 16)]
            dv = dbuf[pl.ds(i * 16, 16)]
            m = (dv >= lo) & (dv < lo + NPT)
            cs = jnp.minimum(cnt, EMAX - 16)
            plsc.store_compressed(osrc.at[pl.ds(cs, 16)], sv, mask=m)
            plsc.store_compressed(odstl.at[pl.ds(cs, 16)], dv - lo, mask=m)
            pop = plsc.all_reduce_population_count(m)
            return cnt + pop[0]

        return lax.fori_loop(0, BB // 16, inner, cnt)

    cnt = lax.fori_loop(0, ET_PAD // BB, blk, 0)
    cntv[...] = jnp.zeros((16,), jnp.int32) + cnt

    ones = jnp.zeros((16,), jnp.int32) + 1

    def cgrp(g, _):
        d16 = odstl[pl.ds(g * 16, 16)]
        plsc.addupdate_scatter(cnts, [d16], ones)
        return 0

    lax.fori_loop(0, EMAX // 16, cgrp, 0)

    def og(g, carry):
        c16 = cnts[pl.ds(g * 16, 16)]
        incl = plsc.cumsum(c16)
        off16 = incl - c16 + carry
        offb[pl.ds(g * 16, 16)] = off16
        cnts[pl.ds(g * 16, 16)] = off16
        return carry + incl[15]

    lax.fori_loop(0, (NPT + 16) // 16, og, 0)

    def sg(g, _):
        s16 = osrc[pl.ds(g * 16, 16)]
        d16 = odstl[pl.ds(g * 16, 16)]
        for q in range(16):
            dl = d16[q]
            cur = cnts[pl.ds(dl, 16)]
            p = cur[0]
            cnts[pl.ds(dl, 16)] = cur + one0i
            row_s = ssrc[pl.ds(p, 16)]
            ssrc[pl.ds(p, 16)] = jnp.where(lane == 0, s16[q], row_s)
            row_d = sdst[pl.ds(p, 16)]
            sdst[pl.ds(p, 16)] = jnp.where(lane == 0, dl, row_d)
        return 0

    lax.fori_loop(0, EMAX // 16, sg, 0)

    pltpu.sync_copy(ssrc.at[pl.ds(0, EMAX)], esrc_hbm.at[w])
    pltpu.sync_copy(sdst.at[pl.ds(0, EMAX)], edstl_hbm.at[w])
    pltpu.sync_copy(cntv, ecnt_hbm.at[w])
    pltpu.sync_copy(offb, eoff_hbm.at[w])


@functools.partial(
    pl.kernel,
    out_type=jax.ShapeDtypeStruct((NCHUNK, NP, CW), jnp.float32),
    mesh=_mesh,
    compiler_params=pltpu.CompilerParams(needs_layout_passes=False),
    scratch_types=[
        pltpu.VMEM((NP,), jnp.float32),
        pltpu.VMEM((NP,), jnp.float32),
        pltpu.VMEM((EMAX,), jnp.int32),
        pltpu.VMEM((EMAX,), jnp.int32),
        pltpu.VMEM((NPT + 16,), jnp.int32),
        pltpu.VMEM((EMAX + 16,), jnp.float32),
        pltpu.VMEM((NPT + 16,), jnp.float32),
        pltpu.VMEM((NPT + 16,), jnp.float32),
        pltpu.VMEM((NPT, CW), jnp.float32),
        pltpu.VMEM((2, GB, CW), jnp.float32),
        pltpu.VMEM((CW,), jnp.float32),
        pltpu.VMEM((16,), jnp.int32),
        pltpu.SemaphoreType.DMA((2,)),
    ],
)
def _aggregate(h8_hbm, asT_hbm, esrc_hbm, edstl_hbm, ecnt_hbm, eoff_hbm,
               b8_hbm, out8_hbm, asb, adb, srcb, dstlb, offb, wbuf, den, inv,
               acc, grow, bbuf, cntv, gsem):
    w = _wid()
    lo = w * NPT
    pltpu.sync_copy(esrc_hbm.at[w], srcb)
    pltpu.sync_copy(edstl_hbm.at[w], dstlb)
    pltpu.sync_copy(ecnt_hbm.at[w], cntv)
    pltpu.sync_copy(eoff_hbm.at[w], offb)
    cnt = cntv[pl.ds(0, 16)][0]

    def za(i, _):
        r, v = i // 8, i % 8
        acc[r, pl.ds(v * 16, 16)] = jnp.zeros((16,), jnp.float32)
        return 0

    lax.fori_loop(0, NPT * 8, za, 0)

    def zd(i, _):
        den[pl.ds(i * 16, 16)] = jnp.zeros((16,), jnp.float32)
        return 0

    lax.fori_loop(0, NPT // 16 + 1, zd, 0)

    lane = lax.iota(jnp.int32, 16)
    one0 = (lane == 0).astype(jnp.float32)

    def head_body(hd, _):
        pltpu.sync_copy(asT_hbm.at[hd], asb)
        pltpu.sync_copy(asT_hbm.at[hd + 4], adb)

        def mx(i, mv):
            return jnp.maximum(mv, asb[pl.ds(i * 16, 16)])

        mv = lax.fori_loop(0, NP // 16, mx,
                           jnp.full((16,), -3e38, jnp.float32))
        Ms = jnp.max(mv)

        def wcomp(i, _):
            sv = srcb[pl.ds(i * 16, 16)]
            dvl = jnp.minimum(dstlb[pl.ds(i * 16, 16)], NPT - 1)
            a_s = plsc.load_gather(asb, [sv])
            a_d = plsc.load_gather(adb, [dvl + lo])
            z = a_s + a_d
            e = jnp.where(z >= 0, z, 0.2 * z)
            zc = a_d + Ms
            cc = jnp.where(zc >= 0, zc, 0.2 * zc)
            wv = jnp.exp(e - cc)
            msk = (lane + i * 16) < cnt
            wv = jnp.where(msk, wv, 0.0)
            wbuf[pl.ds(i * 16, 16)] = wv
            plsc.addupdate_scatter(den, [dvl], wv)
            return 0

        lax.fori_loop(0, EMAX // 16, wcomp, 0)

        def iv(i, _):
            dv = den[pl.ds(i * 16, 16)]
            inv[pl.ds(i * 16, 16)] = 1.0 / (dv + 1e-16)
            den[pl.ds(i * 16, 16)] = jnp.zeros((16,), jnp.float32)
            return 0

        lax.fori_loop(0, NPT // 16 + 1, iv, 0)

        for sub in range(2):
            cidx = 2 * hd + sub
            nb = (cnt + GB - 1) // GB

            def gstart(j, par):
                pltpu.async_copy(
                    h8_hbm.at[cidx].at[srcb.at[pl.ds(j * GB, GB)]],
                    grow.at[par], gsem.at[par])

            def gwait(j, par):
                pltpu.make_async_copy(
                    h8_hbm.at[cidx].at[srcb.at[pl.ds(j * GB, GB)]],
                    grow.at[par], gsem.at[par]).wait()

            @pl.when(nb > 0)
            def _prime():
                gstart(0, 0)

            def eblk2(jj, carry):
                for par in range(2):
                    j = 2 * jj + par
                    kpos0, r0 = carry

                    def _proc(carry_in):
                        @pl.when(j + 1 < nb)
                        def _start_next():
                            gstart(j + 1, 1 - par)

                        gwait(j, par)
                        growp = grow.at[par]
                        blk_end = jnp.minimum((j + 1) * GB, cnt)
                        base = j * GB

                        def wcond(st):
                            return st[0] < blk_end

                        def wbody(st):
                            kpos, r = st
                            o16 = offb[pl.ds(r, 16)]
                            run_end = o16[1]
                            ke = jnp.minimum(run_end, blk_end)

                            def edge(k, a8):
                                w16 = wbuf[pl.ds(k, 16)]
                                gk = k - base
                                return tuple(
                                    a8[v] + growp[gk, pl.ds(v * 16, 16)]
                                    * w16[0]
                                    for v in range(8))

                            z = jnp.zeros((16,), jnp.float32)
                            a8 = lax.fori_loop(kpos, ke, edge,
                                               (z,) * 8)
                            for v in range(8):
                                acc[r, pl.ds(v * 16, 16)] = (
                                    acc[r, pl.ds(v * 16, 16)] + a8[v])
                            r2 = jnp.where(ke == run_end, r + 1, r)
                            return (ke, r2)

                        return lax.while_loop(wcond, wbody, carry_in)

                    proc = lax.cond(j < nb, _proc,
                                    lambda c: c, (kpos0, r0))
                    carry = proc
                return carry

            lax.fori_loop(0, (EMAX // GB + 1) // 2, eblk2, (0, 0))

            pltpu.sync_copy(b8_hbm.at[cidx], bbuf)

            def rw(r, _):
                ivr = inv[pl.ds(r, 16)][0]
                for v in range(8):
                    o = acc[r, pl.ds(v * 16, 16)] * ivr + bbuf[pl.ds(v * 16, 16)]
                    acc[r, pl.ds(v * 16, 16)] = jnp.maximum(o, 0.0)
                return 0

            lax.fori_loop(0, NPT, rw, 0)
            pltpu.sync_copy(acc, out8_hbm.at[cidx, pl.ds(lo, NPT)])

            def za2(i, _):
                r, v = i // 8, i % 8
                acc[r, pl.ds(v * 16, 16)] = jnp.zeros((16,), jnp.float32)
                return 0

            lax.fori_loop(0, NPT * 8, za2, 0)
        return 0

    lax.fori_loop(0, HEADS, head_body, 0)


def _asel(af, hdiv):
    ki = lax.broadcasted_iota(jnp.int32, (CH, HEADS), 0)
    hi = lax.broadcasted_iota(jnp.int32, (CH, HEADS), 1)
    return jnp.where((ki // hdiv) == hi, af[:, None], 0.0)


def _mm0_body(x_ref, w_ref, asf_ref, adf_ref, h8_ref, asT_ref):
    x2 = x_ref[...].astype(jnp.bfloat16)
    w2 = w_ref[...].astype(jnp.bfloat16)
    res = jnp.dot(x2, w2, preferred_element_type=jnp.float32)
    for ci in range(NCHUNK):
        h8_ref[ci] = res[:, ci * CW:(ci + 1) * CW]
    asel_s = _asel(asf_ref[0], HID)
    asel_d = _asel(adf_ref[0], HID)
    wa_s = jnp.dot(w2, asel_s, preferred_element_type=jnp.float32)
    wa_d = jnp.dot(w2, asel_d, preferred_element_type=jnp.float32)
    dn = (((0,), (1,)), ((), ()))
    as_t = lax.dot_general(wa_s, x2, dn, preferred_element_type=jnp.float32)
    ad_t = lax.dot_general(wa_d, x2, dn, preferred_element_type=jnp.float32)
    asT_ref[...] = jnp.concatenate([as_t, ad_t], axis=0)


def _encoder_mm0(x, W, asf, adf):
    bn = 1024
    return pl.pallas_call(
        _mm0_body,
        grid=(NP // bn,),
        in_specs=[
            pl.BlockSpec((bn, 256), lambda i: (i, 0)),
            pl.BlockSpec((256, CH), lambda i: (0, 0)),
            pl.BlockSpec((1, CH), lambda i: (0, 0)),
            pl.BlockSpec((1, CH), lambda i: (0, 0)),
        ],
        out_specs=[
            pl.BlockSpec((NCHUNK, bn, CW), lambda i: (0, i, 0)),
            pl.BlockSpec((2 * HEADS, bn), lambda i: (0, i)),
        ],
        out_shape=[
            jax.ShapeDtypeStruct((NCHUNK, NP, CW), jnp.float32),
            jax.ShapeDtypeStruct((2 * HEADS, NP), jnp.float32),
        ],
    )(x, W, asf, adf)


def _mm_body(x_ref, w_ref, asf_ref, adf_ref, h8_ref, asT_ref, sc_ref):
    kc = pl.program_id(1)
    x2 = x_ref[0].astype(jnp.bfloat16)
    w2 = w_ref[0].astype(jnp.bfloat16)
    res = jnp.dot(x2, w2, preferred_element_type=jnp.float32)
    asel_s = _asel(asf_ref[0], HID)
    asel_d = _asel(adf_ref[0], HID)
    wa_s = jnp.dot(w2, asel_s, preferred_element_type=jnp.float32)
    wa_d = jnp.dot(w2, asel_d, preferred_element_type=jnp.float32)
    dn = (((0,), (1,)), ((), ()))
    as_t = lax.dot_general(wa_s, x2, dn, preferred_element_type=jnp.float32)
    ad_t = lax.dot_general(wa_d, x2, dn, preferred_element_type=jnp.float32)
    upd = jnp.concatenate([as_t, ad_t], axis=0)

    @pl.when(kc == 0)
    def _init():
        sc_ref[...] = res
        asT_ref[...] = upd

    @pl.when(kc > 0)
    def _accum():
        sc_ref[...] = sc_ref[...] + res
        asT_ref[...] = asT_ref[...] + upd

    @pl.when(kc == NCHUNK - 1)
    def _emit():
        tot = sc_ref[...]
        for ci in range(NCHUNK):
            h8_ref[ci] = tot[:, ci * CW:(ci + 1) * CW]


def _encoder_mm(h8_in, W8, asf, adf):
    bn = 1024
    return pl.pallas_call(
        _mm_body,
        grid=(NP // bn, NCHUNK),
        in_specs=[
            pl.BlockSpec((1, bn, CW), lambda i, k: (k, i, 0)),
            pl.BlockSpec((1, CW, CH), lambda i, k: (k, 0, 0)),
            pl.BlockSpec((1, CH), lambda i, k: (0, 0)),
            pl.BlockSpec((1, CH), lambda i, k: (0, 0)),
        ],
        out_specs=[
            pl.BlockSpec((NCHUNK, bn, CW), lambda i, k: (0, i, 0)),
            pl.BlockSpec((2 * HEADS, bn), lambda i, k: (0, i)),
        ],
        out_shape=[
            jax.ShapeDtypeStruct((NCHUNK, NP, CW), jnp.float32),
            jax.ShapeDtypeStruct((2 * HEADS, NP), jnp.float32),
        ],
        scratch_shapes=[pltpu.VMEM((bn, CH), jnp.float32)],
    )(h8_in, W8, asf, adf)


def _pool_body(h_ref, batch_ref, pooled_ref, counts_ref):
    c = pl.program_id(0)
    i = pl.program_id(1)
    bt = batch_ref[0, 0]
    oh = (bt[None, :] ==
          lax.broadcasted_iota(jnp.int32, (NUM_GRAPHS, 1024), 0)
          ).astype(jnp.float32)
    part = jnp.dot(oh, h_ref[0], preferred_element_type=jnp.float32)

    @pl.when(i == 0)
    def _init():
        pooled_ref[0] = part

    @pl.when(i > 0)
    def _accum():
        pooled_ref[0] = pooled_ref[0] + part

    @pl.when(c == 0)
    def _cnt():
        cp = jnp.sum(oh, axis=1)

        @pl.when(i == 0)
        def _ci():
            counts_ref[0] = cp

        @pl.when(i > 0)
        def _ca():
            counts_ref[0] = counts_ref[0] + cp


def _pool(h8, batch_r):
    bn = 1024
    return pl.pallas_call(
        _pool_body,
        grid=(NCHUNK, NP // bn),
        in_specs=[
            pl.BlockSpec((1, bn, CW), lambda c, i: (c, i, 0)),
            pl.BlockSpec((1, 1, bn), lambda c, i: (i, 0, 0)),
        ],
        out_specs=[
            pl.BlockSpec((1, NUM_GRAPHS, CW), lambda c, i: (c, 0, 0)),
            pl.BlockSpec((1, NUM_GRAPHS), lambda c, i: (0, 0)),
        ],
        out_shape=[
            jax.ShapeDtypeStruct((NCHUNK, NUM_GRAPHS, CW), jnp.float32),
            jax.ShapeDtypeStruct((1, NUM_GRAPHS), jnp.float32),
        ],
    )(h8, batch_r)


def _fc_body(pooled_ref, counts_ref, fcw_ref, fcb_ref, out_ref):
    inv = 1.0 / jnp.clip(counts_ref[0], 1.0, None)
    acc = jnp.zeros((NUM_GRAPHS, 10), jnp.float32)
    for c in range(NCHUNK):
        acc = acc + jnp.dot(pooled_ref[c] * inv[:, None], fcw_ref[c],
                            preferred_element_type=jnp.float32)
    out_ref[...] = acc + fcb_ref[0][None, :]


def _fc(pooled8, counts, fcw8, fcb):
    return pl.pallas_call(
        _fc_body,
        in_specs=[
            pl.BlockSpec((NCHUNK, NUM_GRAPHS, CW), lambda: (0, 0, 0)),
            pl.BlockSpec((1, NUM_GRAPHS), lambda: (0, 0)),
            pl.BlockSpec((NCHUNK, CW, 10), lambda: (0, 0, 0)),
            pl.BlockSpec((1, 10), lambda: (0, 0)),
        ],
        out_specs=pl.BlockSpec((NUM_GRAPHS, 10), lambda: (0, 0)),
        out_shape=jax.ShapeDtypeStruct((NUM_GRAPHS, 10), jnp.float32),
    )(pooled8, counts, fcw8, fcb)


def kernel(x, edge_index, batch, W0, a_src0, a_dst0, b0, W1, a_src1, a_dst1,
           b1, W2, a_src2, a_dst2, b2, fc_W, fc_b):
    N = x.shape[0]
    loop = jnp.arange(N, dtype=jnp.int32)
    src = jnp.concatenate([edge_index[0].astype(jnp.int32), loop])
    dst = jnp.concatenate([edge_index[1].astype(jnp.int32), loop])
    src_p = jnp.pad(src, (0, ET_PAD - ET))
    dst_p = jnp.pad(dst, (0, ET_PAD - ET), constant_values=1 << 30)

    esrc, edstl, ecnt, eoff = _bucket(src_p, dst_p)

    x_p = jnp.pad(x, ((0, NP - N), (0, 0)))
    h8, asT = _encoder_mm0(x_p, W0, a_src0.reshape(1, CH),
                           a_dst0.reshape(1, CH))
    h8 = _aggregate(h8, asT, esrc, edstl, ecnt, eoff,
                    b0.reshape(NCHUNK, CW))
    for (W, a_s, a_d, b) in ((W1, a_src1, a_dst1, b1),
                             (W2, a_src2, a_dst2, b2)):
        h8, asT = _encoder_mm(h8, W.reshape(NCHUNK, CW, CH),
                              a_s.reshape(1, CH), a_d.reshape(1, CH))
        h8 = _aggregate(h8, asT, esrc, edstl, ecnt, eoff,
                        b.reshape(NCHUNK, CW))

    batch_r = jnp.pad(batch.astype(jnp.int32), (0, NP - N),
                      constant_values=NUM_GRAPHS).reshape(NP // 1024, 1, 1024)
    pooled8, counts = _pool(h8, batch_r)
    return _fc(pooled8, counts, fc_W.reshape(NCHUNK, CW, 10),
               fc_b.reshape(1, 10))


# speedup vs baseline: 19.1302x; 1.0413x over previous
import functools

import jax
import jax.numpy as jnp
import numpy as np
from jax import lax
from jax.experimental import pallas as pl
from jax.experimental.pallas import tpu as pltpu
from jax.experimental.pallas import tpu_sc as plsc

N_NODES = 10000
NP = 10240
HID = 256
HEADS = 4
CH = HEADS * HID
NCHUNK = 8
CW = 128
NUM_GRAPHS = 64

TILES = 32
NPT = NP // TILES
EMAX = 6400
ET = 170000
BB = 2048
ET_PAD = 84 * BB
GB = 128

_PERM = np.arange(CH).reshape(-1, 2, 16).transpose(0, 2, 1).reshape(-1)

_mesh = plsc.VectorSubcoreMesh(core_axis_name="c", subcore_axis_name="s")


def _wid():
    return lax.axis_index("s") * 2 + lax.axis_index("c")


@functools.partial(
    pl.kernel,
    out_type=[
        jax.ShapeDtypeStruct((TILES, EMAX), jnp.int32),
        jax.ShapeDtypeStruct((TILES, EMAX), jnp.int32),
        jax.ShapeDtypeStruct((TILES, 16), jnp.int32),
        jax.ShapeDtypeStruct((TILES, NPT + 16), jnp.int32),
    ],
    mesh=_mesh,
    compiler_params=pltpu.CompilerParams(needs_layout_passes=False),
    scratch_types=[
        pltpu.VMEM((BB,), jnp.int32),
        pltpu.VMEM((BB,), jnp.int32),
        pltpu.VMEM((EMAX,), jnp.int32),
        pltpu.VMEM((EMAX,), jnp.int32),
        pltpu.VMEM((EMAX + 16,), jnp.int32),
        pltpu.VMEM((EMAX + 16,), jnp.int32),
        pltpu.VMEM((NPT + 16,), jnp.int32),
        pltpu.VMEM((NPT + 16,), jnp.int32),
        pltpu.VMEM((16,), jnp.int32),
    ],
)
def _bucket(src_hbm, dst_hbm, esrc_hbm, edstl_hbm, ecnt_hbm, eoff_hbm,
            sbuf, dbuf, osrc, odstl, ssrc, sdst, cnts, offb, cntv):
    w = _wid()
    lo = w * NPT
    lane = lax.iota(jnp.int32, 16)
    one0i = (lane == 0).astype(jnp.int32)

    def zi(i, _):
        osrc[pl.ds(i * 16, 16)] = jnp.zeros((16,), jnp.int32)
        odstl[pl.ds(i * 16, 16)] = jnp.zeros((16,), jnp.int32) + NPT
        return 0

    lax.fori_loop(0, EMAX // 16, zi, 0)

    def zc(i, _):
        cnts[pl.ds(i * 16, 16)] = jnp.zeros((16,), jnp.int32)
        return 0

    lax.fori_loop(0, (NPT + 16) // 16, zc, 0)

    def blk(j, cnt):
        pltpu.sync_copy(src_hbm.at[pl.ds(j * BB, BB)], sbuf)
        pltpu.sync_copy(dst_hbm.at[pl.ds(j * BB, BB)], dbuf)

        def inner(i, cnt):
            sv = sbuf[pl.ds(i * 16, 16)]
            dv = dbuf[pl.ds(i * 16, 16)]
            m = (dv >= lo) & (dv < lo + NPT)
            cs = jnp.minimum(cnt, EMAX - 16)
            plsc.store_compressed(osrc.at[pl.ds(cs, 16)], sv, mask=m)
            plsc.store_compressed(odstl.at[pl.ds(cs, 16)], dv - lo, mask=m)
            pop = plsc.all_reduce_population_count(m)
            return cnt + pop[0]

        return lax.fori_loop(0, BB // 16, inner, cnt)

    cnt = lax.fori_loop(0, ET_PAD // BB, blk, 0)
    cntv[...] = jnp.zeros((16,), jnp.int32) + cnt

    ones = jnp.zeros((16,), jnp.int32) + 1

    def cgrp(g, _):
        d16 = odstl[pl.ds(g * 16, 16)]
        plsc.addupdate_scatter(cnts, [d16], ones)
        return 0

    lax.fori_loop(0, EMAX // 16, cgrp, 0)

    def og(g, carry):
        c16 = cnts[pl.ds(g * 16, 16)]
        incl = plsc.cumsum(c16)
        off16 = incl - c16 + carry
        offb[pl.ds(g * 16, 16)] = off16
        cnts[pl.ds(g * 16, 16)] = off16
        return carry + incl[15]

    lax.fori_loop(0, (NPT + 16) // 16, og, 0)

    def sg(g, _):
        s16 = osrc[pl.ds(g * 16, 16)]
        d16 = odstl[pl.ds(g * 16, 16)]
        for q in range(16):
            dl = d16[q]
            cur = cnts[pl.ds(dl, 16)]
            p = cur[0]
            cnts[pl.ds(dl, 16)] = cur + one0i
            row_s = ssrc[pl.ds(p, 16)]
            ssrc[pl.ds(p, 16)] = jnp.where(lane == 0, s16[q], row_s)
            row_d = sdst[pl.ds(p, 16)]
            sdst[pl.ds(p, 16)] = jnp.where(lane == 0, dl, row_d)
        return 0

    lax.fori_loop(0, EMAX // 16, sg, 0)

    pltpu.sync_copy(ssrc.at[pl.ds(0, EMAX)], esrc_hbm.at[w])
    pltpu.sync_copy(sdst.at[pl.ds(0, EMAX)], edstl_hbm.at[w])
    pltpu.sync_copy(cntv, ecnt_hbm.at[w])
    pltpu.sync_copy(offb, eoff_hbm.at[w])


@functools.partial(
    pl.kernel,
    out_type=jax.ShapeDtypeStruct((NCHUNK, NP, CW), jnp.float32),
    mesh=_mesh,
    compiler_params=pltpu.CompilerParams(needs_layout_passes=False),
    scratch_types=[
        pltpu.VMEM((NP,), jnp.float32),
        pltpu.VMEM((NP,), jnp.float32),
        pltpu.VMEM((EMAX,), jnp.int32),
        pltpu.VMEM((EMAX,), jnp.int32),
        pltpu.VMEM((NPT + 16,), jnp.int32),
        pltpu.VMEM((EMAX + 16,), jnp.float32),
        pltpu.VMEM((NPT + 16,), jnp.float32),
        pltpu.VMEM((NPT + 16,), jnp.float32),
        pltpu.VMEM((NPT, CW), jnp.float32),
        pltpu.VMEM((2, GB, CW), jnp.float32),
        pltpu.VMEM((CW,), jnp.float32),
        pltpu.VMEM((16,), jnp.int32),
        pltpu.SemaphoreType.DMA((2,)),
    ],
)
def _aggregate(h8_hbm, asT_hbm, esrc_hbm, edstl_hbm, ecnt_hbm, eoff_hbm,
               b8_hbm, out8_hbm, asb, adb, srcb, dstlb, offb, wbuf, den, inv,
               acc, grow, bbuf, cntv, gsem):
    w = _wid()
    lo = w * NPT
    pltpu.sync_copy(esrc_hbm.at[w], srcb)
    pltpu.sync_copy(edstl_hbm.at[w], dstlb)
    pltpu.sync_copy(ecnt_hbm.at[w], cntv)
    pltpu.sync_copy(eoff_hbm.at[w], offb)
    cnt = cntv[pl.ds(0, 16)][0]

    def za(i, _):
        r, v = i // 8, i % 8
        acc[r, pl.ds(v * 16, 16)] = jnp.zeros((16,), jnp.float32)
        return 0

    lax.fori_loop(0, NPT * 8, za, 0)

    def zd(i, _):
        den[pl.ds(i * 16, 16)] = jnp.zeros((16,), jnp.float32)
        return 0

    lax.fori_loop(0, NPT // 16 + 1, zd, 0)

    lane = lax.iota(jnp.int32, 16)
    one0 = (lane == 0).astype(jnp.float32)

    def head_body(hd, _):
        pltpu.sync_copy(asT_hbm.at[hd], asb)
        pltpu.sync_copy(asT_hbm.at[hd + 4], adb)

        def mx(i, mv):
            return jnp.maximum(mv, asb[pl.ds(i * 16, 16)])

        mv = lax.fori_loop(0, NP // 16, mx,
                           jnp.full((16,), -3e38, jnp.float32))
        Ms = jnp.max(mv)

        def wcomp(i, _):
            sv = srcb[pl.ds(i * 16, 16)]
            dvl = jnp.minimum(dstlb[pl.ds(i * 16, 16)], NPT - 1)
            a_s = plsc.load_gather(asb, [sv])
            a_d = plsc.load_gather(adb, [dvl + lo])
            z = a_s + a_d
            e = jnp.where(z >= 0, z, 0.2 * z)
            zc = a_d + Ms
            cc = jnp.where(zc >= 0, zc, 0.2 * zc)
            wv = jnp.exp(e - cc)
            msk = (lane + i * 16) < cnt
            wv = jnp.where(msk, wv, 0.0)
            wbuf[pl.ds(i * 16, 16)] = wv
            plsc.addupdate_scatter(den, [dvl], wv)
            return 0

        lax.fori_loop(0, EMAX // 16, wcomp, 0)

        def iv(i, _):
            dv = den[pl.ds(i * 16, 16)]
            ivv = jnp.where(dv > 0, 1.0 / (dv + 1e-16), 0.0)
            inv[pl.ds(i * 16, 16)] = ivv
            den[pl.ds(i * 16, 16)] = jnp.zeros((16,), jnp.float32)
            return 0

        lax.fori_loop(0, NPT // 16 + 1, iv, 0)

        for sub in range(2):
            cidx = 2 * hd + sub
            nb = (cnt + GB - 1) // GB

            def gstart(j, par):
                pltpu.async_copy(
                    h8_hbm.at[cidx].at[srcb.at[pl.ds(j * GB, GB)]],
                    grow.at[par], gsem.at[par])

            def gwait(j, par):
                pltpu.make_async_copy(
                    h8_hbm.at[cidx].at[srcb.at[pl.ds(j * GB, GB)]],
                    grow.at[par], gsem.at[par]).wait()

            @pl.when(nb > 0)
            def _prime():
                gstart(0, 0)

            def eblk2(jj, carry):
                for par in range(2):
                    j = 2 * jj + par
                    kpos0, r0 = carry

                    def _proc(carry_in):
                        @pl.when(j + 1 < nb)
                        def _start_next():
                            gstart(j + 1, 1 - par)

                        gwait(j, par)
                        growp = grow.at[par]
                        blk_end = jnp.minimum((j + 1) * GB, cnt)
                        base = j * GB

                        def wcond(st):
                            return st[0] < blk_end

                        def wbody(st):
                            kpos, r = st
                            o16 = offb[pl.ds(r, 16)]
                            run_end = o16[1]
                            first = kpos == o16[0]
                            ke = jnp.minimum(run_end, blk_end)

                            def edge(k, a8):
                                w16 = wbuf[pl.ds(k, 16)]
                                gk = k - base
                                return tuple(
                                    a8[v] + growp[gk, pl.ds(v * 16, 16)]
                                    * w16[0]
                                    for v in range(8))

                            z = jnp.zeros((16,), jnp.float32)
                            a8 = lax.fori_loop(kpos, ke, edge,
                                               (z,) * 8)
                            for v in range(8):
                                acc[r, pl.ds(v * 16, 16)] = jnp.where(
                                    first, a8[v],
                                    acc[r, pl.ds(v * 16, 16)] + a8[v])
                            r2 = jnp.where(ke == run_end, r + 1, r)
                            return (ke, r2)

                        return lax.while_loop(wcond, wbody, carry_in)

                    proc = lax.cond(j < nb, _proc,
                                    lambda c: c, (kpos0, r0))
                    carry = proc
                return carry

            lax.fori_loop(0, (EMAX // GB + 1) // 2, eblk2, (0, 0))

            pltpu.sync_copy(b8_hbm.at[cidx], bbuf)

            def rw(r, _):
                ivr = inv[pl.ds(r, 16)][0]
                for v in range(8):
                    o = acc[r, pl.ds(v * 16, 16)] * ivr + bbuf[pl.ds(v * 16, 16)]
                    acc[r, pl.ds(v * 16, 16)] = jnp.maximum(o, 0.0)
                return 0

            lax.fori_loop(0, NPT, rw, 0)
            pltpu.sync_copy(acc, out8_hbm.at[cidx, pl.ds(lo, NPT)])
        return 0

    lax.fori_loop(0, HEADS, head_body, 0)


def _asel(af, hdiv):
    ki = lax.broadcasted_iota(jnp.int32, (CH, HEADS), 0)
    hi = lax.broadcasted_iota(jnp.int32, (CH, HEADS), 1)
    return jnp.where((ki // hdiv) == hi, af[:, None], 0.0)


def _mm0_body(x_ref, w_ref, asf_ref, adf_ref, h8_ref, asT_ref):
    x2 = x_ref[...].astype(jnp.bfloat16)
    w2 = w_ref[...].astype(jnp.bfloat16)
    res = jnp.dot(x2, w2, preferred_element_type=jnp.float32)
    for ci in range(NCHUNK):
        h8_ref[ci] = res[:, ci * CW:(ci + 1) * CW]
    asel_s = _asel(asf_ref[0], HID)
    asel_d = _asel(adf_ref[0], HID)
    wa_s = jnp.dot(w2, asel_s, preferred_element_type=jnp.float32)
    wa_d = jnp.dot(w2, asel_d, preferred_element_type=jnp.float32)
    dn = (((0,), (1,)), ((), ()))
    as_t = lax.dot_general(wa_s, x2, dn, preferred_element_type=jnp.float32)
    ad_t = lax.dot_general(wa_d, x2, dn, preferred_element_type=jnp.float32)
    asT_ref[...] = jnp.concatenate([as_t, ad_t], axis=0)


def _encoder_mm0(x, W, asf, adf):
    bn = 1024
    return pl.pallas_call(
        _mm0_body,
        grid=(NP // bn,),
        in_specs=[
            pl.BlockSpec((bn, 256), lambda i: (i, 0)),
            pl.BlockSpec((256, CH), lambda i: (0, 0)),
            pl.BlockSpec((1, CH), lambda i: (0, 0)),
            pl.BlockSpec((1, CH), lambda i: (0, 0)),
        ],
        out_specs=[
            pl.BlockSpec((NCHUNK, bn, CW), lambda i: (0, i, 0)),
            pl.BlockSpec((2 * HEADS, bn), lambda i: (0, i)),
        ],
        out_shape=[
            jax.ShapeDtypeStruct((NCHUNK, NP, CW), jnp.float32),
            jax.ShapeDtypeStruct((2 * HEADS, NP), jnp.float32),
        ],
    )(x, W, asf, adf)


def _mm_body(x_ref, w_ref, asf_ref, adf_ref, h8_ref, asT_ref, sc_ref):
    kc = pl.program_id(1)
    x2 = x_ref[0].astype(jnp.bfloat16)
    w2 = w_ref[0].astype(jnp.bfloat16)
    res = jnp.dot(x2, w2, preferred_element_type=jnp.float32)
    asel_s = _asel(asf_ref[0], HID)
    asel_d = _asel(adf_ref[0], HID)
    wa_s = jnp.dot(w2, asel_s, preferred_element_type=jnp.float32)
    wa_d = jnp.dot(w2, asel_d, preferred_element_type=jnp.float32)
    dn = (((0,), (1,)), ((), ()))
    as_t = lax.dot_general(wa_s, x2, dn, preferred_element_type=jnp.float32)
    ad_t = lax.dot_general(wa_d, x2, dn, preferred_element_type=jnp.float32)
    upd = jnp.concatenate([as_t, ad_t], axis=0)

    @pl.when(kc == 0)
    def _init():
        sc_ref[...] = res
        asT_ref[...] = upd

    @pl.when(kc > 0)
    def _accum():
        sc_ref[...] = sc_ref[...] + res
        asT_ref[...] = asT_ref[...] + upd

    @pl.when(kc == NCHUNK - 1)
    def _emit():
        tot = sc_ref[...]
        for ci in range(NCHUNK):
            h8_ref[ci] = tot[:, ci * CW:(ci + 1) * CW]


def _encoder_mm(h8_in, W8, asf, adf):
    bn = 1024
    return pl.pallas_call(
        _mm_body,
        grid=(NP // bn, NCHUNK),
        in_specs=[
            pl.BlockSpec((1, bn, CW), lambda i, k: (k, i, 0)),
            pl.BlockSpec((1, CW, CH), lambda i, k: (k, 0, 0)),
            pl.BlockSpec((1, CH), lambda i, k: (0, 0)),
            pl.BlockSpec((1, CH), lambda i, k: (0, 0)),
        ],
        out_specs=[
            pl.BlockSpec((NCHUNK, bn, CW), lambda i, k: (0, i, 0)),
            pl.BlockSpec((2 * HEADS, bn), lambda i, k: (0, i)),
        ],
        out_shape=[
            jax.ShapeDtypeStruct((NCHUNK, NP, CW), jnp.float32),
            jax.ShapeDtypeStruct((2 * HEADS, NP), jnp.float32),
        ],
        scratch_shapes=[pltpu.VMEM((bn, CH), jnp.float32)],
    )(h8_in, W8, asf, adf)


def _pool_body(h_ref, batch_ref, pooled_ref, counts_ref):
    c = pl.program_id(0)
    i = pl.program_id(1)
    bt = batch_ref[0, 0]
    oh = (bt[None, :] ==
          lax.broadcasted_iota(jnp.int32, (NUM_GRAPHS, 1024), 0)
          ).astype(jnp.float32)
    part = jnp.dot(oh, h_ref[0], preferred_element_type=jnp.float32)

    @pl.when(i == 0)
    def _init():
        pooled_ref[0] = part

    @pl.when(i > 0)
    def _accum():
        pooled_ref[0] = pooled_ref[0] + part

    @pl.when(c == 0)
    def _cnt():
        cp = jnp.sum(oh, axis=1)

        @pl.when(i == 0)
        def _ci():
            counts_ref[0] = cp

        @pl.when(i > 0)
        def _ca():
            counts_ref[0] = counts_ref[0] + cp


def _pool(h8, batch_r):
    bn = 1024
    return pl.pallas_call(
        _pool_body,
        grid=(NCHUNK, NP // bn),
        in_specs=[
            pl.BlockSpec((1, bn, CW), lambda c, i: (c, i, 0)),
            pl.BlockSpec((1, 1, bn), lambda c, i: (i, 0, 0)),
        ],
        out_specs=[
            pl.BlockSpec((1, NUM_GRAPHS, CW), lambda c, i: (c, 0, 0)),
            pl.BlockSpec((1, NUM_GRAPHS), lambda c, i: (0, 0)),
        ],
        out_shape=[
            jax.ShapeDtypeStruct((NCHUNK, NUM_GRAPHS, CW), jnp.float32),
            jax.ShapeDtypeStruct((1, NUM_GRAPHS), jnp.float32),
        ],
    )(h8, batch_r)


def _fc_body(pooled_ref, counts_ref, fcw_ref, fcb_ref, out_ref):
    inv = 1.0 / jnp.clip(counts_ref[0], 1.0, None)
    acc = jnp.zeros((NUM_GRAPHS, 10), jnp.float32)
    for c in range(NCHUNK):
        acc = acc + jnp.dot(pooled_ref[c] * inv[:, None], fcw_ref[c],
                            preferred_element_type=jnp.float32)
    out_ref[...] = acc + fcb_ref[0][None, :]


def _fc(pooled8, counts, fcw8, fcb):
    return pl.pallas_call(
        _fc_body,
        in_specs=[
            pl.BlockSpec((NCHUNK, NUM_GRAPHS, CW), lambda: (0, 0, 0)),
            pl.BlockSpec((1, NUM_GRAPHS), lambda: (0, 0)),
            pl.BlockSpec((NCHUNK, CW, 10), lambda: (0, 0, 0)),
            pl.BlockSpec((1, 10), lambda: (0, 0)),
        ],
        out_specs=pl.BlockSpec((NUM_GRAPHS, 10), lambda: (0, 0)),
        out_shape=jax.ShapeDtypeStruct((NUM_GRAPHS, 10), jnp.float32),
    )(pooled8, counts, fcw8, fcb)


def kernel(x, edge_index, batch, W0, a_src0, a_dst0, b0, W1, a_src1, a_dst1,
           b1, W2, a_src2, a_dst2, b2, fc_W, fc_b):
    N = x.shape[0]
    loop = jnp.arange(N, dtype=jnp.int32)
    src = jnp.concatenate([edge_index[0].astype(jnp.int32), loop])
    dst = jnp.concatenate([edge_index[1].astype(jnp.int32), loop])
    src_p = jnp.pad(src, (0, ET_PAD - ET))
    dst_p = jnp.pad(dst, (0, ET_PAD - ET), constant_values=1 << 30)

    esrc, edstl, ecnt, eoff = _bucket(src_p, dst_p)

    x_p = jnp.pad(x, ((0, NP - N), (0, 0)))
    h8, asT = _encoder_mm0(x_p, W0, a_src0.reshape(1, CH),
                           a_dst0.reshape(1, CH))
    h8 = _aggregate(h8, asT, esrc, edstl, ecnt, eoff,
                    b0.reshape(NCHUNK, CW))
    for (W, a_s, a_d, b) in ((W1, a_src1, a_dst1, b1),
                             (W2, a_src2, a_dst2, b2)):
        h8, asT = _encoder_mm(h8, W.reshape(NCHUNK, CW, CH),
                              a_s.reshape(1, CH), a_d.reshape(1, CH))
        h8 = _aggregate(h8, asT, esrc, edstl, ecnt, eoff,
                        b.reshape(NCHUNK, CW))

    batch_r = jnp.pad(batch.astype(jnp.int32), (0, NP - N),
                      constant_values=NUM_GRAPHS).reshape(NP // 1024, 1, 1024)
    pooled8, counts = _pool(h8, batch_r)
    return _fc(pooled8, counts, fc_W.reshape(NCHUNK, CW, 10),
               fc_b.reshape(1, 10))


# speedup vs baseline: 19.1646x; 1.0018x over previous
import functools

import jax
import jax.numpy as jnp
import numpy as np
from jax import lax
from jax.experimental import pallas as pl
from jax.experimental.pallas import tpu as pltpu
from jax.experimental.pallas import tpu_sc as plsc

N_NODES = 10000
NP = 10240
HID = 256
HEADS = 4
CH = HEADS * HID
NCHUNK = 8
CW = 128
NUM_GRAPHS = 64

TILES = 32
NPT = NP // TILES
EMAX = 6400
ET = 170000
BB = 2048
ET_PAD = 84 * BB
GB = 128

_PERM = np.arange(CH).reshape(-1, 2, 16).transpose(0, 2, 1).reshape(-1)

_mesh = plsc.VectorSubcoreMesh(core_axis_name="c", subcore_axis_name="s")


def _wid():
    return lax.axis_index("s") * 2 + lax.axis_index("c")


@functools.partial(
    pl.kernel,
    out_type=[
        jax.ShapeDtypeStruct((TILES, EMAX), jnp.int32),
        jax.ShapeDtypeStruct((TILES, EMAX), jnp.int32),
        jax.ShapeDtypeStruct((TILES, 16), jnp.int32),
        jax.ShapeDtypeStruct((TILES, NPT + 16), jnp.int32),
    ],
    mesh=_mesh,
    compiler_params=pltpu.CompilerParams(needs_layout_passes=False),
    scratch_types=[
        pltpu.VMEM((BB,), jnp.int32),
        pltpu.VMEM((BB,), jnp.int32),
        pltpu.VMEM((EMAX,), jnp.int32),
        pltpu.VMEM((EMAX,), jnp.int32),
        pltpu.VMEM((EMAX + 16,), jnp.int32),
        pltpu.VMEM((EMAX + 16,), jnp.int32),
        pltpu.VMEM((NPT + 16,), jnp.int32),
        pltpu.VMEM((NPT + 16,), jnp.int32),
        pltpu.VMEM((16,), jnp.int32),
    ],
)
def _bucket(src_hbm, dst_hbm, esrc_hbm, edstl_hbm, ecnt_hbm, eoff_hbm,
            sbuf, dbuf, osrc, odstl, ssrc, sdst, cnts, offb, cntv):
    w = _wid()
    lo = w * NPT
    lane = lax.iota(jnp.int32, 16)
    one0i = (lane == 0).astype(jnp.int32)

    def zi(i, _):
        osrc[pl.ds(i * 16, 16)] = jnp.zeros((16,), jnp.int32)
        odstl[pl.ds(i * 16, 16)] = jnp.zeros((16,), jnp.int32) + NPT
        return 0

    lax.fori_loop(0, EMAX // 16, zi, 0)

    def zc(i, _):
        cnts[pl.ds(i * 16, 16)] = jnp.zeros((16,), jnp.int32)
        return 0

    lax.fori_loop(0, (NPT + 16) // 16, zc, 0)

    def blk(j, cnt):
        pltpu.sync_copy(src_hbm.at[pl.ds(j * BB, BB)], sbuf)
        pltpu.sync_copy(dst_hbm.at[pl.ds(j * BB, BB)], dbuf)

        def inner(i, cnt):
            sv = sbuf[pl.ds(i * 16, 16)]
            dv = dbuf[pl.ds(i * 16, 16)]
            m = (dv >= lo) & (dv < lo + NPT)
            cs = jnp.minimum(cnt, EMAX - 16)
            plsc.store_compressed(osrc.at[pl.ds(cs, 16)], sv, mask=m)
            plsc.store_compressed(odstl.at[pl.ds(cs, 16)], dv - lo, mask=m)
            pop = plsc.all_reduce_population_count(m)
            return cnt + pop[0]

        return lax.fori_loop(0, BB // 16, inner, cnt)

    cnt = lax.fori_loop(0, ET_PAD // BB, blk, 0)
    cntv[...] = jnp.zeros((16,), jnp.int32) + cnt

    ones = jnp.zeros((16,), jnp.int32) + 1

    def cgrp(g, _):
        d16 = odstl[pl.ds(g * 16, 16)]
        plsc.addupdate_scatter(cnts, [d16], ones)
        return 0

    lax.fori_loop(0, EMAX // 16, cgrp, 0)

    def og(g, carry):
        c16 = cnts[pl.ds(g * 16, 16)]
        incl = plsc.cumsum(c16)
        off16 = incl - c16 + carry
        offb[pl.ds(g * 16, 16)] = off16
        cnts[pl.ds(g * 16, 16)] = off16
        return carry + incl[15]

    lax.fori_loop(0, (NPT + 16) // 16, og, 0)

    def sg(g, _):
        s16 = osrc[pl.ds(g * 16, 16)]
        d16 = odstl[pl.ds(g * 16, 16)]
        for q in range(16):
            dl = d16[q]
            cur = cnts[pl.ds(dl, 16)]
            p = cur[0]
            cnts[pl.ds(dl, 16)] = cur + one0i
            row_s = ssrc[pl.ds(p, 16)]
            ssrc[pl.ds(p, 16)] = jnp.where(lane == 0, s16[q], row_s)
            row_d = sdst[pl.ds(p, 16)]
            sdst[pl.ds(p, 16)] = jnp.where(lane == 0, dl, row_d)
        return 0

    lax.fori_loop(0, EMAX // 16, sg, 0)

    pltpu.sync_copy(ssrc.at[pl.ds(0, EMAX)], esrc_hbm.at[w])
    pltpu.sync_copy(sdst.at[pl.ds(0, EMAX)], edstl_hbm.at[w])
    pltpu.sync_copy(cntv, ecnt_hbm.at[w])
    pltpu.sync_copy(offb, eoff_hbm.at[w])


@functools.partial(
    pl.kernel,
    out_type=jax.ShapeDtypeStruct((NCHUNK, NP, CW), jnp.float32),
    mesh=_mesh,
    compiler_params=pltpu.CompilerParams(needs_layout_passes=False),
    scratch_types=[
        pltpu.VMEM((NP,), jnp.float32),
        pltpu.VMEM((NP,), jnp.float32),
        pltpu.VMEM((EMAX,), jnp.int32),
        pltpu.VMEM((EMAX,), jnp.int32),
        pltpu.VMEM((NPT + 16,), jnp.int32),
        pltpu.VMEM((EMAX + 16,), jnp.float32),
        pltpu.VMEM((NPT + 16,), jnp.float32),
        pltpu.VMEM((NPT + 16,), jnp.float32),
        pltpu.VMEM((NPT, CW), jnp.float32),
        pltpu.VMEM((2, GB, CW), jnp.float32),
        pltpu.VMEM((CW,), jnp.float32),
        pltpu.VMEM((16,), jnp.int32),
        pltpu.SemaphoreType.DMA((2,)),
    ],
)
def _aggregate(h8_hbm, asT_hbm, esrc_hbm, edstl_hbm, ecnt_hbm, eoff_hbm,
               b8_hbm, out8_hbm, asb, adb, srcb, dstlb, offb, wbuf, den, inv,
               acc, grow, bbuf, cntv, gsem):
    w = _wid()
    lo = w * NPT
    pltpu.sync_copy(esrc_hbm.at[w], srcb)
    pltpu.sync_copy(edstl_hbm.at[w], dstlb)
    pltpu.sync_copy(ecnt_hbm.at[w], cntv)
    pltpu.sync_copy(eoff_hbm.at[w], offb)
    cnt = cntv[pl.ds(0, 16)][0]

    def za(i, _):
        r, v = i // 8, i % 8
        acc[r, pl.ds(v * 16, 16)] = jnp.zeros((16,), jnp.float32)
        return 0

    lax.fori_loop(0, NPT * 8, za, 0)

    def zd(i, _):
        den[pl.ds(i * 16, 16)] = jnp.zeros((16,), jnp.float32)
        return 0

    lax.fori_loop(0, NPT // 16 + 1, zd, 0)

    lane = lax.iota(jnp.int32, 16)

    def head_body(hd, _):
        pltpu.sync_copy(asT_hbm.at[hd], asb)
        pltpu.sync_copy(asT_hbm.at[hd + 4], adb)

        def mx(i, mv):
            return jnp.maximum(mv, asb[pl.ds(i * 16, 16)])

        mv = lax.fori_loop(0, NP // 16, mx,
                           jnp.full((16,), -3e38, jnp.float32))
        Ms = jnp.max(mv)

        def wcomp(i, _):
            sv = srcb[pl.ds(i * 16, 16)]
            dvl = jnp.minimum(dstlb[pl.ds(i * 16, 16)], NPT - 1)
            a_s = plsc.load_gather(asb, [sv])
            a_d = plsc.load_gather(adb, [dvl + lo])
            z = a_s + a_d
            e = jnp.where(z >= 0, z, 0.2 * z)
            zc = a_d + Ms
            cc = jnp.where(zc >= 0, zc, 0.2 * zc)
            wv = jnp.exp(e - cc)
            msk = (lane + i * 16) < cnt
            wv = jnp.where(msk, wv, 0.0)
            wbuf[pl.ds(i * 16, 16)] = wv
            plsc.addupdate_scatter(den, [dvl], wv)
            return 0

        lax.fori_loop(0, EMAX // 16, wcomp, 0)

        def iv(i, _):
            dv = den[pl.ds(i * 16, 16)]
            ivv = jnp.where(dv > 0, 1.0 / (dv + 1e-16), 0.0)
            inv[pl.ds(i * 16, 16)] = ivv
            den[pl.ds(i * 16, 16)] = jnp.zeros((16,), jnp.float32)
            return 0

        lax.fori_loop(0, NPT // 16 + 1, iv, 0)

        for sub in range(2):
            cidx = 2 * hd + sub
            nb = (cnt + GB - 1) // GB

            def gstart(j, par):
                pltpu.async_copy(
                    h8_hbm.at[cidx].at[srcb.at[pl.ds(j * GB, GB)]],
                    grow.at[par], gsem.at[par])

            def gwait(j, par):
                pltpu.make_async_copy(
                    h8_hbm.at[cidx].at[srcb.at[pl.ds(j * GB, GB)]],
                    grow.at[par], gsem.at[par]).wait()

            @pl.when(nb > 0)
            def _prime():
                gstart(0, 0)

            def eblk2(jj, carry):
                for par in range(2):
                    j = 2 * jj + par
                    kpos0, r0 = carry

                    def _proc(carry_in):
                        @pl.when(j + 1 < nb)
                        def _start_next():
                            gstart(j + 1, 1 - par)

                        gwait(j, par)
                        growp = grow.at[par]
                        blk_end = jnp.minimum((j + 1) * GB, cnt)
                        base = j * GB

                        def wcond(st):
                            return st[0] < blk_end

                        def wbody(st):
                            kpos, r = st
                            o16 = offb[pl.ds(r, 16)]
                            run_end = o16[1]
                            first = kpos == o16[0]
                            ke = jnp.minimum(run_end, blk_end)

                            def edge(k, a8):
                                w16 = wbuf[pl.ds(k, 16)]
                                gk = k - base
                                return tuple(
                                    a8[v] + growp[gk, pl.ds(v * 16, 16)]
                                    * w16[0]
                                    for v in range(8))

                            z = jnp.zeros((16,), jnp.float32)
                            a8 = lax.fori_loop(kpos, ke, edge,
                                               (z,) * 8)
                            for v in range(8):
                                acc[r, pl.ds(v * 16, 16)] = jnp.where(
                                    first, a8[v],
                                    acc[r, pl.ds(v * 16, 16)] + a8[v])
                            r2 = jnp.where(ke == run_end, r + 1, r)
                            return (ke, r2)

                        return lax.while_loop(wcond, wbody, carry_in)

                    proc = lax.cond(j < nb, _proc,
                                    lambda c: c, (kpos0, r0))
                    carry = proc
                return carry

            lax.fori_loop(0, (EMAX // GB + 1) // 2, eblk2, (0, 0))

            pltpu.sync_copy(b8_hbm.at[cidx], bbuf)

            def rw(r, _):
                ivr = inv[pl.ds(r, 16)][0]
                for v in range(8):
                    o = acc[r, pl.ds(v * 16, 16)] * ivr + bbuf[pl.ds(v * 16, 16)]
                    acc[r, pl.ds(v * 16, 16)] = jnp.maximum(o, 0.0)
                return 0

            lax.fori_loop(0, NPT, rw, 0)
            pltpu.sync_copy(acc, out8_hbm.at[cidx, pl.ds(lo, NPT)])
        return 0

    lax.fori_loop(0, HEADS, head_body, 0)


def _asel(af, hdiv):
    ki = lax.broadcasted_iota(jnp.int32, (CH, HEADS), 0)
    hi = lax.broadcasted_iota(jnp.int32, (CH, HEADS), 1)
    return jnp.where((ki // hdiv) == hi, af[:, None], 0.0)


def _mm0_body(x_ref, w_ref, asf_ref, adf_ref, h8_ref, asT_ref):
    x2 = x_ref[...].astype(jnp.bfloat16)
    w2 = w_ref[...].astype(jnp.bfloat16)
    res = jnp.dot(x2, w2, preferred_element_type=jnp.float32)
    for ci in range(NCHUNK):
        h8_ref[ci] = res[:, ci * CW:(ci + 1) * CW]
    asel_s = _asel(asf_ref[0], HID)
    asel_d = _asel(adf_ref[0], HID)
    wa_s = jnp.dot(w2, asel_s, preferred_element_type=jnp.float32)
    wa_d = jnp.dot(w2, asel_d, preferred_element_type=jnp.float32)
    dn = (((0,), (1,)), ((), ()))
    as_t = lax.dot_general(wa_s, x2, dn, preferred_element_type=jnp.float32)
    ad_t = lax.dot_general(wa_d, x2, dn, preferred_element_type=jnp.float32)
    asT_ref[...] = jnp.concatenate([as_t, ad_t], axis=0)


def _encoder_mm0(x, W, asf, adf):
    bn = 1024
    return pl.pallas_call(
        _mm0_body,
        grid=(NP // bn,),
        in_specs=[
            pl.BlockSpec((bn, 256), lambda i: (i, 0)),
            pl.BlockSpec((256, CH), lambda i: (0, 0)),
            pl.BlockSpec((1, CH), lambda i: (0, 0)),
            pl.BlockSpec((1, CH), lambda i: (0, 0)),
        ],
        out_specs=[
            pl.BlockSpec((NCHUNK, bn, CW), lambda i: (0, i, 0)),
            pl.BlockSpec((2 * HEADS, bn), lambda i: (0, i)),
        ],
        out_shape=[
            jax.ShapeDtypeStruct((NCHUNK, NP, CW), jnp.float32),
            jax.ShapeDtypeStruct((2 * HEADS, NP), jnp.float32),
        ],
    )(x, W, asf, adf)


def _mm_body(x_ref, w_ref, asf_ref, adf_ref, h8_ref, asT_ref, sc_ref):
    kc = pl.program_id(1)
    x2 = x_ref[0].astype(jnp.bfloat16)
    w2 = w_ref[0].astype(jnp.bfloat16)
    res = jnp.dot(x2, w2, preferred_element_type=jnp.float32)
    asel_s = _asel(asf_ref[0], HID)
    asel_d = _asel(adf_ref[0], HID)
    wa_s = jnp.dot(w2, asel_s, preferred_element_type=jnp.float32)
    wa_d = jnp.dot(w2, asel_d, preferred_element_type=jnp.float32)
    dn = (((0,), (1,)), ((), ()))
    as_t = lax.dot_general(wa_s, x2, dn, preferred_element_type=jnp.float32)
    ad_t = lax.dot_general(wa_d, x2, dn, preferred_element_type=jnp.float32)
    upd = jnp.concatenate([as_t, ad_t], axis=0)

    @pl.when(kc == 0)
    def _init():
        sc_ref[...] = res
        asT_ref[...] = upd

    @pl.when(kc > 0)
    def _accum():
        sc_ref[...] = sc_ref[...] + res
        asT_ref[...] = asT_ref[...] + upd

    @pl.when(kc == NCHUNK - 1)
    def _emit():
        tot = sc_ref[...]
        for ci in range(NCHUNK):
            h8_ref[ci] = tot[:, ci * CW:(ci + 1) * CW]


def _encoder_mm(h8_in, W8, asf, adf):
    bn = 1024
    return pl.pallas_call(
        _mm_body,
        grid=(NP // bn, NCHUNK),
        in_specs=[
            pl.BlockSpec((1, bn, CW), lambda i, k: (k, i, 0)),
            pl.BlockSpec((1, CW, CH), lambda i, k: (k, 0, 0)),
            pl.BlockSpec((1, CH), lambda i, k: (0, 0)),
            pl.BlockSpec((1, CH), lambda i, k: (0, 0)),
        ],
        out_specs=[
            pl.BlockSpec((NCHUNK, bn, CW), lambda i, k: (0, i, 0)),
            pl.BlockSpec((2 * HEADS, bn), lambda i, k: (0, i)),
        ],
        out_shape=[
            jax.ShapeDtypeStruct((NCHUNK, NP, CW), jnp.float32),
            jax.ShapeDtypeStruct((2 * HEADS, NP), jnp.float32),
        ],
        scratch_shapes=[pltpu.VMEM((bn, CH), jnp.float32)],
    )(h8_in, W8, asf, adf)


def _pool_body(h_ref, batch_ref, pooled_ref, counts_ref):
    c = pl.program_id(0)
    i = pl.program_id(1)
    bt = batch_ref[0, 0]
    oh = (bt[None, :] ==
          lax.broadcasted_iota(jnp.int32, (NUM_GRAPHS, 1024), 0)
          ).astype(jnp.float32)
    part = jnp.dot(oh, h_ref[0], preferred_element_type=jnp.float32)

    @pl.when(i == 0)
    def _init():
        pooled_ref[0] = part

    @pl.when(i > 0)
    def _accum():
        pooled_ref[0] = pooled_ref[0] + part

    @pl.when(c == 0)
    def _cnt():
        cp = jnp.sum(oh, axis=1)

        @pl.when(i == 0)
        def _ci():
            counts_ref[0] = cp

        @pl.when(i > 0)
        def _ca():
            counts_ref[0] = counts_ref[0] + cp


def _pool(h8, batch_r):
    bn = 1024
    return pl.pallas_call(
        _pool_body,
        grid=(NCHUNK, NP // bn),
        in_specs=[
            pl.BlockSpec((1, bn, CW), lambda c, i: (c, i, 0)),
            pl.BlockSpec((1, 1, bn), lambda c, i: (i, 0, 0)),
        ],
        out_specs=[
            pl.BlockSpec((1, NUM_GRAPHS, CW), lambda c, i: (c, 0, 0)),
            pl.BlockSpec((1, NUM_GRAPHS), lambda c, i: (0, 0)),
        ],
        out_shape=[
            jax.ShapeDtypeStruct((NCHUNK, NUM_GRAPHS, CW), jnp.float32),
            jax.ShapeDtypeStruct((1, NUM_GRAPHS), jnp.float32),
        ],
    )(h8, batch_r)


def _fc_body(pooled_ref, counts_ref, fcw_ref, fcb_ref, out_ref):
    inv = 1.0 / jnp.clip(counts_ref[0], 1.0, None)
    acc = jnp.zeros((NUM_GRAPHS, 10), jnp.float32)
    for c in range(NCHUNK):
        acc = acc + jnp.dot(pooled_ref[c] * inv[:, None], fcw_ref[c],
                            preferred_element_type=jnp.float32)
    out_ref[...] = acc + fcb_ref[0][None, :]


def _fc(pooled8, counts, fcw8, fcb):
    return pl.pallas_call(
        _fc_body,
        in_specs=[
            pl.BlockSpec((NCHUNK, NUM_GRAPHS, CW), lambda: (0, 0, 0)),
            pl.BlockSpec((1, NUM_GRAPHS), lambda: (0, 0)),
            pl.BlockSpec((NCHUNK, CW, 10), lambda: (0, 0, 0)),
            pl.BlockSpec((1, 10), lambda: (0, 0)),
        ],
        out_specs=pl.BlockSpec((NUM_GRAPHS, 10), lambda: (0, 0)),
        out_shape=jax.ShapeDtypeStruct((NUM_GRAPHS, 10), jnp.float32),
    )(pooled8, counts, fcw8, fcb)


def kernel(x, edge_index, batch, W0, a_src0, a_dst0, b0, W1, a_src1, a_dst1,
           b1, W2, a_src2, a_dst2, b2, fc_W, fc_b):
    N = x.shape[0]
    loop = jnp.arange(N, dtype=jnp.int32)
    src = jnp.concatenate([edge_index[0].astype(jnp.int32), loop])
    dst = jnp.concatenate([edge_index[1].astype(jnp.int32), loop])
    src_p = jnp.pad(src, (0, ET_PAD - ET))
    dst_p = jnp.pad(dst, (0, ET_PAD - ET), constant_values=1 << 30)

    esrc, edstl, ecnt, eoff = _bucket(src_p, dst_p)

    x_p = jnp.pad(x, ((0, NP - N), (0, 0)))
    h8, asT = _encoder_mm0(x_p, W0, a_src0.reshape(1, CH),
                           a_dst0.reshape(1, CH))
    h8 = _aggregate(h8, asT, esrc, edstl, ecnt, eoff,
                    b0.reshape(NCHUNK, CW))
    for (W, a_s, a_d, b) in ((W1, a_src1, a_dst1, b1),
                             (W2, a_src2, a_dst2, b2)):
        h8, asT = _encoder_mm(h8, W.reshape(NCHUNK, CW, CH),
                              a_s.reshape(1, CH), a_d.reshape(1, CH))
        h8 = _aggregate(h8, asT, esrc, edstl, ecnt, eoff,
                        b.reshape(NCHUNK, CW))

    batch_r = jnp.pad(batch.astype(jnp.int32), (0, NP - N),
                      constant_values=NUM_GRAPHS).reshape(NP // 1024, 1, 1024)
    pooled8, counts = _pool(h8, batch_r)
    return _fc(pooled8, counts, fc_W.reshape(NCHUNK, CW, 10),
               fc_b.reshape(1, 10))
